# baseline (speedup 1.0000x reference)
import functools

import jax
import jax.numpy as jnp
from jax import lax
from jax.experimental import pallas as pl
from jax.experimental.pallas import tpu as pltpu
from jax.experimental.pallas import tpu_sc as plsc

HID = 128
PPAD = 16
NC, NS = 2, 16
NW = NC * NS
C = 80


def _silu(x):
  return x * jax.nn.sigmoid(x)


def _sc_gather_body(epw, nchunk, h_hbm, posp_hbm, row_hbm, col_hbm,
                    hr_hbm, hc_hbm, pr_hbm, pc_hbm,
                    idx_r, idx_c, hbuf_r, hbuf_c, pbuf_r, pbuf_c, sem):
  wid = lax.axis_index("s") * NC + lax.axis_index("c")
  base = wid * epw

  def chunk(i, carry):
    off = base + i * C
    pltpu.sync_copy(row_hbm.at[pl.ds(off, C)], idx_r)
    pltpu.sync_copy(col_hbm.at[pl.ds(off, C)], idx_c)
    d1 = pltpu.async_copy(h_hbm.at[idx_r], hbuf_r, sem)
    d2 = pltpu.async_copy(h_hbm.at[idx_c], hbuf_c, sem)
    d3 = pltpu.async_copy(posp_hbm.at[idx_r], pbuf_r, sem)
    d4 = pltpu.async_copy(posp_hbm.at[idx_c], pbuf_c, sem)
    d1.wait()
    d2.wait()
    d3.wait()
    d4.wait()
    pltpu.sync_copy(hbuf_r, hr_hbm.at[pl.ds(off, C)])
    pltpu.sync_copy(hbuf_c, hc_hbm.at[pl.ds(off, C)])
    pltpu.sync_copy(pbuf_r, pr_hbm.at[pl.ds(off, C)])
    pltpu.sync_copy(pbuf_c, pc_hbm.at[pl.ds(off, C)])
    return carry

  lax.fori_loop(0, nchunk, chunk, 0)


def _sc_gather(h32, posp, row, col):
  e = row.shape[0]
  hw = h32.shape[1]
  epw = e // NW
  nchunk = epw // C
  mesh = plsc.VectorSubcoreMesh(core_axis_name="c", subcore_axis_name="s",
                                num_cores=NC, num_subcores=NS)
  f = pl.kernel(
      functools.partial(_sc_gather_body, epw, nchunk),
      compiler_params=pltpu.CompilerParams(use_tc_tiling_on_sc=False),
      out_type=(
          jax.ShapeDtypeStruct((e, hw), jnp.int32),
          jax.ShapeDtypeStruct((e, hw), jnp.int32),
          jax.ShapeDtypeStruct((e, PPAD), jnp.float32),
          jax.ShapeDtypeStruct((e, PPAD), jnp.float32),
      ),
      mesh=mesh,
      scratch_types=(
          pltpu.VMEM((C,), jnp.int32),
          pltpu.VMEM((C,), jnp.int32),
          pltpu.VMEM((C, hw), jnp.int32),
          pltpu.VMEM((C, hw), jnp.int32),
          pltpu.VMEM((C, PPAD), jnp.float32),
          pltpu.VMEM((C, PPAD), jnp.float32),
          pltpu.SemaphoreType.DMA,
      ),
  )
  return f(h32, posp, row, col)


def _sc_scatter_body(epw, nchunk, row_hbm, m2_hbm, cwd_hbm, zm_hbm, zc_hbm,
                     aggm_hbm, aggc_hbm,
                     idx, mbuf, cbuf, accm, accc, sem):
  cid = lax.axis_index("c")
  sid = lax.axis_index("s")
  wid = sid * NC + cid

  @pl.when(sid == 0)
  def _zero():
    pltpu.sync_copy(zm_hbm, accm)
    pltpu.sync_copy(zc_hbm, accc)

  plsc.subcore_barrier()

  base = wid * epw

  def chunk(i, carry):
    off = base + i * C
    pltpu.sync_copy(row_hbm.at[pl.ds(off, C)], idx)
    pltpu.sync_copy(m2_hbm.at[pl.ds(off, C)], mbuf)
    pltpu.sync_copy(cwd_hbm.at[pl.ds(off, C)], cbuf)
    pltpu.sync_copy(mbuf, accm.at[idx], add=True)
    pltpu.sync_copy(cbuf, accc.at[idx], add=True)
    return carry

  lax.fori_loop(0, nchunk, chunk, 0)

  plsc.subcore_barrier()

  @pl.when(sid == 0)
  def _flush():
    pltpu.sync_copy(accm, aggm_hbm.at[cid])
    pltpu.sync_copy(accc, aggc_hbm.at[cid])


def _sc_scatter(row, m2, cwd, n):
  e = row.shape[0]
  epw = e // NW
  nchunk = epw // C
  zm = jnp.zeros((n, HID), jnp.float32)
  zc = jnp.zeros((n, PPAD), jnp.float32)
  mesh = plsc.VectorSubcoreMesh(core_axis_name="c", subcore_axis_name="s",
                                num_cores=NC, num_subcores=NS)
  f = pl.kernel(
      functools.partial(_sc_scatter_body, epw, nchunk),
      compiler_params=pltpu.CompilerParams(use_tc_tiling_on_sc=False),
      out_type=(
          jax.ShapeDtypeStruct((NC, n, HID), jnp.float32),
          jax.ShapeDtypeStruct((NC, n, PPAD), jnp.float32),
      ),
      mesh=mesh,
      scratch_types=(
          pltpu.VMEM((C,), jnp.int32),
          pltpu.VMEM((C, HID), jnp.float32),
          pltpu.VMEM((C, PPAD), jnp.float32),
          pltpu.VMEM_SHARED((n, HID), jnp.float32),
          pltpu.VMEM_SHARED((n, PPAD), jnp.float32),
          pltpu.SemaphoreType.DMA,
      ),
  )
  return f(row, m2, cwd, zm, zc)


def _tc_edge_body(hr, hc, pr, pc, w1a, w1b, w1d, b1, w2, b2, wc1, bc1, wc2,
                  m2_out, cwd_out):
  diff = pr[...] - pc[...]
  d2 = jnp.sum(diff * diff, axis=1, keepdims=True)
  dist = jnp.sqrt(d2 + 1e-8)
  x = jnp.dot(hr[...], w1a[...], preferred_element_type=jnp.float32)
  x = x + jnp.dot(hc[...], w1b[...], preferred_element_type=jnp.float32)
  x = x + dist * w1d[...] + b1[...]
  m1 = _silu(x).astype(jnp.bfloat16)
  y = jnp.dot(m1, w2[...], preferred_element_type=jnp.float32) + b2[...]
  m2 = _silu(y)
  m2b = m2.astype(jnp.bfloat16)
  z = _silu(jnp.dot(m2b, wc1[...], preferred_element_type=jnp.float32)
            + bc1[...])
  cw = jnp.sum(z * wc2[...], axis=1, keepdims=True)
  m2_out[...] = m2
  cwd_out[...] = (cw / (dist + 1e-8)) * diff


def _tc_edge(hr, hc, pr, pc, w1a, w1b, w1d, b1, w2, b2, wc1, bc1, wc2):
  e = hr.shape[0]
  be = 1280
  grid = (e // be,)
  blk = lambda r, c: pl.BlockSpec((r, c), lambda i: (i, 0))
  wblk = lambda r, c: pl.BlockSpec((r, c), lambda i: (0, 0))
  hr = hr.astype(jnp.bfloat16)
  hc = hc.astype(jnp.bfloat16)
  return pl.pallas_call(
      _tc_edge_body,
      grid=grid,
      in_specs=[
          blk(be, HID), blk(be, HID), blk(be, PPAD), blk(be, PPAD),
          wblk(HID, HID), wblk(HID, HID), wblk(1, HID), wblk(1, HID),
          wblk(HID, HID), wblk(1, HID),
          wblk(HID, HID), wblk(1, HID), wblk(1, HID),
      ],
      out_specs=[blk(be, HID), blk(be, PPAD)],
      out_shape=[
          jax.ShapeDtypeStruct((e, HID), jnp.float32),
          jax.ShapeDtypeStruct((e, PPAD), jnp.float32),
      ],
  )(hr, hc, pr, pc, w1a, w1b, w1d, b1, w2, b2, wc1, bc1, wc2)


def _tc_node_body(h, posp, aggm, aggc, wn1a, wn1b, bn1, wn2, bn2, g, b,
                  h_out, posp_out):
  agg = aggm[0] + aggm[1]
  x = jnp.dot(h[...], wn1a[...], preferred_element_type=jnp.float32)
  x = x + jnp.dot(agg, wn1b[...], preferred_element_type=jnp.float32)
  x = _silu(x + bn1[...])
  hupd = jnp.dot(x, wn2[...], preferred_element_type=jnp.float32) + bn2[...]
  y = h[...] + hupd
  mu = jnp.mean(y, axis=1, keepdims=True)
  var = jnp.mean((y - mu) * (y - mu), axis=1, keepdims=True)
  h_out[...] = (y - mu) * jax.lax.rsqrt(var + 1e-5) * g[...] + b[...]
  posp_out[...] = posp[...] + aggc[0] + aggc[1]


def _tc_node(h, posp, aggm, aggc, wn1a, wn1b, bn1, wn2, bn2, g, b):
  n = h.shape[0]
  bn = 1000
  grid = (n // bn,)
  blk = lambda r, c: pl.BlockSpec((r, c), lambda i: (i, 0))
  wblk = lambda r, c: pl.BlockSpec((r, c), lambda i: (0, 0))
  blk3 = lambda r, c: pl.BlockSpec((NC, r, c), lambda i: (0, i, 0))
  return pl.pallas_call(
      _tc_node_body,
      grid=grid,
      in_specs=[
          blk(bn, HID), blk(bn, PPAD), blk3(bn, HID), blk3(bn, PPAD),
          wblk(HID, HID), wblk(HID, HID), wblk(1, HID),
          wblk(HID, HID), wblk(1, HID), wblk(1, HID), wblk(1, HID),
      ],
      out_specs=[blk(bn, HID), blk(bn, PPAD)],
      out_shape=[
          jax.ShapeDtypeStruct((n, HID), jnp.float32),
          jax.ShapeDtypeStruct((n, PPAD), jnp.float32),
      ],
  )(h, posp, aggm, aggc, wn1a, wn1b, bn1, wn2, bn2, g, b)


@jax.jit
def kernel(h, pos, edge_index, W_e1, b_e1, W_e2, b_e2, W_n1, b_n1, W_n2,
           b_n2, W_c1, b_c1, W_c2, ln_gamma, ln_beta):
  n = h.shape[0]
  row = edge_index[0].astype(jnp.int32)
  col = edge_index[1].astype(jnp.int32)
  posp = jnp.zeros((n, PPAD), jnp.float32).at[:, :3].set(pos)

  h32 = jax.lax.bitcast_convert_type(
      h.astype(jnp.bfloat16).reshape(n, HID // 2, 2), jnp.int32)
  hr32, hc32, pr, pc = _sc_gather(h32, posp, row, col)
  e = hr32.shape[0]
  hr = jax.lax.bitcast_convert_type(hr32, jnp.bfloat16).reshape(e, HID)
  hc = jax.lax.bitcast_convert_type(hc32, jnp.bfloat16).reshape(e, HID)

  w1a = W_e1[:HID].astype(jnp.bfloat16)
  w1b = W_e1[HID:2 * HID].astype(jnp.bfloat16)
  w1d = W_e1[2 * HID].reshape(1, HID)
  m2, cwd = _tc_edge(hr, hc, pr, pc, w1a, w1b, w1d, b_e1.reshape(1, HID),
                     W_e2.astype(jnp.bfloat16), b_e2.reshape(1, HID),
                     W_c1.astype(jnp.bfloat16), b_c1.reshape(1, HID),
                     W_c2.reshape(1, HID))

  aggm, aggc = _sc_scatter(row, m2, cwd, n)

  h_out, posp_out = _tc_node(h, posp, aggm, aggc, W_n1[:HID], W_n1[HID:],
                             b_n1.reshape(1, HID), W_n2,
                             b_n2.reshape(1, HID), ln_gamma.reshape(1, HID),
                             ln_beta.reshape(1, HID))
  return h_out, posp_out[:, :3]


# speedup vs baseline: 1.7110x; 1.7110x over previous
import functools

import jax
import jax.numpy as jnp
from jax import lax
from jax.experimental import pallas as pl
from jax.experimental.pallas import tpu as pltpu
from jax.experimental.pallas import tpu_sc as plsc

HID = 128
PPAD = 16
NC, NS = 2, 16
NW = NC * NS
C = 80


def _silu(x):
  return x * jax.nn.sigmoid(x)


def _sc_gather_body(epw, nchunk, h_hbm, posp_hbm, row_hbm, col_hbm,
                    hr_hbm, hc_hbm, pr_hbm, pc_hbm,
                    idx_r, idx_c, hbuf_r, hbuf_c, pbuf_r, pbuf_c, sem):
  wid = lax.axis_index("s") * NC + lax.axis_index("c")
  base = wid * epw

  def chunk(i, carry):
    off = base + i * C
    pltpu.sync_copy(row_hbm.at[pl.ds(off, C)], idx_r)
    pltpu.sync_copy(col_hbm.at[pl.ds(off, C)], idx_c)
    d1 = pltpu.async_copy(h_hbm.at[idx_r], hbuf_r, sem)
    d2 = pltpu.async_copy(h_hbm.at[idx_c], hbuf_c, sem)
    d3 = pltpu.async_copy(posp_hbm.at[idx_r], pbuf_r, sem)
    d4 = pltpu.async_copy(posp_hbm.at[idx_c], pbuf_c, sem)
    d1.wait()
    d2.wait()
    d3.wait()
    d4.wait()
    pltpu.sync_copy(hbuf_r, hr_hbm.at[pl.ds(off, C)])
    pltpu.sync_copy(hbuf_c, hc_hbm.at[pl.ds(off, C)])
    pltpu.sync_copy(pbuf_r, pr_hbm.at[pl.ds(off, C)])
    pltpu.sync_copy(pbuf_c, pc_hbm.at[pl.ds(off, C)])
    return carry

  lax.fori_loop(0, nchunk, chunk, 0)


def _sc_gather(h32, posp, row, col):
  e = row.shape[0]
  hw = h32.shape[1]
  epw = e // NW
  nchunk = epw // C
  mesh = plsc.VectorSubcoreMesh(core_axis_name="c", subcore_axis_name="s",
                                num_cores=NC, num_subcores=NS)
  f = pl.kernel(
      functools.partial(_sc_gather_body, epw, nchunk),
      compiler_params=pltpu.CompilerParams(use_tc_tiling_on_sc=False),
      out_type=(
          jax.ShapeDtypeStruct((e, hw), jnp.int32),
          jax.ShapeDtypeStruct((e, hw), jnp.int32),
          jax.ShapeDtypeStruct((e, PPAD), jnp.float32),
          jax.ShapeDtypeStruct((e, PPAD), jnp.float32),
      ),
      mesh=mesh,
      scratch_types=(
          pltpu.VMEM((C,), jnp.int32),
          pltpu.VMEM((C,), jnp.int32),
          pltpu.VMEM((C, hw), jnp.int32),
          pltpu.VMEM((C, hw), jnp.int32),
          pltpu.VMEM((C, PPAD), jnp.float32),
          pltpu.VMEM((C, PPAD), jnp.float32),
          pltpu.SemaphoreType.DMA,
      ),
  )
  return f(h32, posp, row, col)


def _sc_scatter_body(epw, nchunk, row_hbm, m2_hbm, cwd_hbm, zm_hbm, zc_hbm,
                     aggm_hbm, aggc_hbm,
                     idx, mbuf, cbuf, accm, accc, sem):
  cid = lax.axis_index("c")
  sid = lax.axis_index("s")
  wid = sid * NC + cid

  @pl.when(sid == 0)
  def _zero():
    pltpu.sync_copy(zm_hbm, accm)
    pltpu.sync_copy(zc_hbm, accc)

  plsc.subcore_barrier()

  base = wid * epw

  def chunk(i, carry):
    off = base + i * C
    pltpu.sync_copy(row_hbm.at[pl.ds(off, C)], idx)
    pltpu.sync_copy(m2_hbm.at[pl.ds(off, C)], mbuf)
    pltpu.sync_copy(cwd_hbm.at[pl.ds(off, C)], cbuf)
    pltpu.sync_copy(mbuf, accm.at[idx], add=True)
    pltpu.sync_copy(cbuf, accc.at[idx], add=True)
    return carry

  lax.fori_loop(0, nchunk, chunk, 0)

  plsc.subcore_barrier()

  @pl.when(sid == 0)
  def _flush():
    pltpu.sync_copy(accm, aggm_hbm.at[cid])
    pltpu.sync_copy(accc, aggc_hbm.at[cid])


def _sc_scatter(row, m2, cwd, n):
  e = row.shape[0]
  epw = e // NW
  nchunk = epw // C
  zm = jnp.zeros((n, HID), jnp.float32)
  zc = jnp.zeros((n, PPAD), jnp.float32)
  mesh = plsc.VectorSubcoreMesh(core_axis_name="c", subcore_axis_name="s",
                                num_cores=NC, num_subcores=NS)
  f = pl.kernel(
      functools.partial(_sc_scatter_body, epw, nchunk),
      compiler_params=pltpu.CompilerParams(use_tc_tiling_on_sc=False),
      out_type=(
          jax.ShapeDtypeStruct((NC, n, HID), jnp.float32),
          jax.ShapeDtypeStruct((NC, n, PPAD), jnp.float32),
      ),
      mesh=mesh,
      scratch_types=(
          pltpu.VMEM((C,), jnp.int32),
          pltpu.VMEM((C, HID), jnp.float32),
          pltpu.VMEM((C, PPAD), jnp.float32),
          pltpu.VMEM_SHARED((n, HID), jnp.float32),
          pltpu.VMEM_SHARED((n, PPAD), jnp.float32),
          pltpu.SemaphoreType.DMA,
      ),
  )
  return f(row, m2, cwd, zm, zc)


def _unpack_bf16(x32):
  lo = jax.lax.bitcast_convert_type(x32 << 16, jnp.float32)
  hi = jax.lax.bitcast_convert_type(
      x32 & jnp.int32(-65536), jnp.float32)
  return jnp.concatenate([lo, hi], axis=1).astype(jnp.bfloat16)


def _tc_edge_body(hr, hc, pr, pc, w1a, w1d, b1, w2, b2, wc1, bc1, wc2,
                  m2_out, cwd_out):
  diff = pr[...] - pc[...]
  d2 = jnp.sum(diff * diff, axis=1, keepdims=True)
  dist = jnp.sqrt(d2 + 1e-8)
  hb = jnp.concatenate([_unpack_bf16(hr[...]), _unpack_bf16(hc[...])],
                       axis=1)
  x = jnp.dot(hb, w1a[...], preferred_element_type=jnp.float32)
  x = (x + dist * w1d[...] + b1[...]).astype(jnp.bfloat16)
  m1 = _silu(x)
  y = jnp.dot(m1, w2[...], preferred_element_type=jnp.float32) + b2[...]
  m2 = _silu(y)
  m2b = m2.astype(jnp.bfloat16)
  z = _silu((jnp.dot(m2b, wc1[...], preferred_element_type=jnp.float32)
             + bc1[...]).astype(jnp.bfloat16)).astype(jnp.float32)
  cw = jnp.sum(z * wc2[...], axis=1, keepdims=True)
  m2_out[...] = m2
  cwd_out[...] = (cw / (dist + 1e-8)) * diff


def _tc_edge(hr, hc, pr, pc, w1a, w1d, b1, w2, b2, wc1, bc1, wc2):
  e = hr.shape[0]
  be = 1280
  grid = (e // be,)
  blk = lambda r, c: pl.BlockSpec((r, c), lambda i: (i, 0))
  wblk = lambda r, c: pl.BlockSpec((r, c), lambda i: (0, 0))
  return pl.pallas_call(
      _tc_edge_body,
      grid=grid,
      in_specs=[
          blk(be, HID // 2), blk(be, HID // 2), blk(be, PPAD), blk(be, PPAD),
          wblk(2 * HID, HID), wblk(1, HID), wblk(1, HID),
          wblk(HID, HID), wblk(1, HID),
          wblk(HID, HID), wblk(1, HID), wblk(1, HID),
      ],
      out_specs=[blk(be, HID), blk(be, PPAD)],
      out_shape=[
          jax.ShapeDtypeStruct((e, HID), jnp.float32),
          jax.ShapeDtypeStruct((e, PPAD), jnp.float32),
      ],
  )(hr, hc, pr, pc, w1a, w1d, b1, w2, b2, wc1, bc1, wc2)


def _tc_node_body(h, posp, aggm, aggc, wn1a, wn1b, bn1, wn2, bn2, g, b,
                  h_out, posp_out):
  agg = aggm[0] + aggm[1]
  x = jnp.dot(h[...], wn1a[...], preferred_element_type=jnp.float32)
  x = x + jnp.dot(agg, wn1b[...], preferred_element_type=jnp.float32)
  x = _silu(x + bn1[...])
  hupd = jnp.dot(x, wn2[...], preferred_element_type=jnp.float32) + bn2[...]
  y = h[...] + hupd
  mu = jnp.mean(y, axis=1, keepdims=True)
  var = jnp.mean((y - mu) * (y - mu), axis=1, keepdims=True)
  h_out[...] = (y - mu) * jax.lax.rsqrt(var + 1e-5) * g[...] + b[...]
  posp_out[...] = posp[...] + aggc[0] + aggc[1]


def _tc_node(h, posp, aggm, aggc, wn1a, wn1b, bn1, wn2, bn2, g, b):
  n = h.shape[0]
  bn = 1000
  grid = (n // bn,)
  blk = lambda r, c: pl.BlockSpec((r, c), lambda i: (i, 0))
  wblk = lambda r, c: pl.BlockSpec((r, c), lambda i: (0, 0))
  blk3 = lambda r, c: pl.BlockSpec((NC, r, c), lambda i: (0, i, 0))
  return pl.pallas_call(
      _tc_node_body,
      grid=grid,
      in_specs=[
          blk(bn, HID), blk(bn, PPAD), blk3(bn, HID), blk3(bn, PPAD),
          wblk(HID, HID), wblk(HID, HID), wblk(1, HID),
          wblk(HID, HID), wblk(1, HID), wblk(1, HID), wblk(1, HID),
      ],
      out_specs=[blk(bn, HID), blk(bn, PPAD)],
      out_shape=[
          jax.ShapeDtypeStruct((n, HID), jnp.float32),
          jax.ShapeDtypeStruct((n, PPAD), jnp.float32),
      ],
  )(h, posp, aggm, aggc, wn1a, wn1b, bn1, wn2, bn2, g, b)


@jax.jit
def kernel(h, pos, edge_index, W_e1, b_e1, W_e2, b_e2, W_n1, b_n1, W_n2,
           b_n2, W_c1, b_c1, W_c2, ln_gamma, ln_beta):
  n = h.shape[0]
  row = edge_index[0].astype(jnp.int32)
  col = edge_index[1].astype(jnp.int32)
  posp = jnp.zeros((n, PPAD), jnp.float32).at[:, :3].set(pos)

  h32 = jax.lax.bitcast_convert_type(
      h.astype(jnp.bfloat16).reshape(n, HID // 2, 2), jnp.int32)
  hr32, hc32, pr, pc = _sc_gather(h32, posp, row, col)

  wa = W_e1[:HID]
  wb = W_e1[HID:2 * HID]
  w1a = jnp.concatenate([wa[0::2], wa[1::2], wb[0::2], wb[1::2]],
                        axis=0).astype(jnp.bfloat16)
  w1d = W_e1[2 * HID].reshape(1, HID)
  m2, cwd = _tc_edge(hr32, hc32, pr, pc, w1a, w1d, b_e1.reshape(1, HID),
                     W_e2.astype(jnp.bfloat16), b_e2.reshape(1, HID),
                     W_c1.astype(jnp.bfloat16), b_c1.reshape(1, HID),
                     W_c2.reshape(1, HID))

  aggm, aggc = _sc_scatter(row, m2, cwd, n)

  h_out, posp_out = _tc_node(h, posp, aggm, aggc, W_n1[:HID], W_n1[HID:],
                             b_n1.reshape(1, HID), W_n2,
                             b_n2.reshape(1, HID), ln_gamma.reshape(1, HID),
                             ln_beta.reshape(1, HID))
  return h_out, posp_out[:, :3]


# speedup vs baseline: 1.8746x; 1.0956x over previous
import functools

import jax
import jax.numpy as jnp
from jax import lax
from jax.experimental import pallas as pl
from jax.experimental.pallas import tpu as pltpu
from jax.experimental.pallas import tpu_sc as plsc

HID = 128
PPAD = 16
NC, NS = 2, 16
NW = NC * NS
C = 80
L = 16


def _silu(x):
  return x * jax.nn.sigmoid(x)


def _sc_gather_body(epw, nchunk, tbl_hbm, row_hbm, col_hbm,
                    hr_hbm, hc_hbm,
                    idx_r, idx_c, buf_r, buf_c, sem):
  wid = lax.axis_index("s") * NC + lax.axis_index("c")
  base = wid * epw

  def chunk(i, carry):
    off = base + i * C
    pltpu.sync_copy(row_hbm.at[pl.ds(off, C)], idx_r)
    pltpu.sync_copy(col_hbm.at[pl.ds(off, C)], idx_c)
    d1 = pltpu.async_copy(tbl_hbm.at[idx_r], buf_r, sem)
    d2 = pltpu.async_copy(tbl_hbm.at[idx_c], buf_c, sem)
    d1.wait()
    d2.wait()
    pltpu.sync_copy(buf_r, hr_hbm.at[pl.ds(off, C)])
    pltpu.sync_copy(buf_c, hc_hbm.at[pl.ds(off, C)])
    return carry

  lax.fori_loop(0, nchunk, chunk, 0)


def _sc_gather(tbl, row, col):
  e = row.shape[0]
  epw = e // NW
  nchunk = epw // C
  mesh = plsc.VectorSubcoreMesh(core_axis_name="c", subcore_axis_name="s",
                                num_cores=NC, num_subcores=NS)
  f = pl.kernel(
      functools.partial(_sc_gather_body, epw, nchunk),
      out_type=(
          jax.ShapeDtypeStruct((e, HID), jnp.int32),
          jax.ShapeDtypeStruct((e, HID), jnp.int32),
      ),
      mesh=mesh,
      scratch_types=(
          pltpu.VMEM((C,), jnp.int32),
          pltpu.VMEM((C,), jnp.int32),
          pltpu.VMEM((C, HID), jnp.int32),
          pltpu.VMEM((C, HID), jnp.int32),
          pltpu.SemaphoreType.DMA,
      ),
  )
  return f(tbl, row, col)


def _sc_scatter_m_body(epw, nchunk, row_hbm, m2_hbm, zm_hbm, aggm_hbm,
                       idx, mbuf, accm, sem):
  cid = lax.axis_index("c")
  sid = lax.axis_index("s")
  wid = sid * NC + cid

  @pl.when(sid == 0)
  def _zero():
    pltpu.sync_copy(zm_hbm, accm)

  plsc.subcore_barrier()

  base = wid * epw

  def chunk(i, carry):
    off = base + i * C
    pltpu.sync_copy(row_hbm.at[pl.ds(off, C)], idx)
    pltpu.sync_copy(m2_hbm.at[pl.ds(off, C)], mbuf)
    pltpu.sync_copy(mbuf, accm.at[idx], add=True)
    return carry

  lax.fori_loop(0, nchunk, chunk, 0)

  plsc.subcore_barrier()

  @pl.when(sid == 0)
  def _flush():
    pltpu.sync_copy(accm, aggm_hbm.at[cid])


def _sc_scatter_m(row, m2, n):
  e = row.shape[0]
  epw = e // NW
  nchunk = epw // C
  zm = jnp.zeros((n, HID), jnp.float32)
  mesh = plsc.VectorSubcoreMesh(core_axis_name="c", subcore_axis_name="s",
                                num_cores=NC, num_subcores=NS)
  f = pl.kernel(
      functools.partial(_sc_scatter_m_body, epw, nchunk),
      out_type=jax.ShapeDtypeStruct((NC, n, HID), jnp.float32),
      mesh=mesh,
      scratch_types=(
          pltpu.VMEM((C,), jnp.int32),
          pltpu.VMEM((C, HID), jnp.float32),
          pltpu.VMEM_SHARED((n, HID), jnp.float32),
          pltpu.SemaphoreType.DMA,
      ),
  )
  return f(row, m2, zm)


def _sc_scatter_c_body(epw, nchunk, row_hbm, col_hbm, s_hbm,
                       px_hbm, py_hbm, pz_hbm, zc_hbm, aggc_hbm,
                       idx_r, idx_c, sbuf, cbuf, px_v, py_v, pz_v, accc, sem):
  cid = lax.axis_index("c")
  sid = lax.axis_index("s")
  wid = sid * NC + cid

  pltpu.sync_copy(px_hbm, px_v)
  pltpu.sync_copy(py_hbm, py_v)
  pltpu.sync_copy(pz_hbm, pz_v)

  @pl.when(sid == 0)
  def _zero():
    pltpu.sync_copy(zc_hbm, accc)

  pltpu.sync_copy(zc_hbm.at[pl.ds(0, C)], cbuf)

  plsc.subcore_barrier()

  base = wid * epw
  lane = lax.iota(jnp.int32, L)
  zero16 = jnp.zeros((L,), jnp.int32)

  def chunk(i, carry):
    off = base + i * C
    pltpu.sync_copy(row_hbm.at[pl.ds(off, C)], idx_r)
    pltpu.sync_copy(col_hbm.at[pl.ds(off, C)], idx_c)
    pltpu.sync_copy(s_hbm.at[pl.ds(off, C)], sbuf)
    for j in range(C // L):
      ir = idx_r[pl.ds(j * L, L)]
      ic = idx_c[pl.ds(j * L, L)]
      sv = sbuf[pl.ds(j * L, L)]
      dx = plsc.load_gather(px_v, [ir]) - plsc.load_gather(px_v, [ic])
      dy = plsc.load_gather(py_v, [ir]) - plsc.load_gather(py_v, [ic])
      dz = plsc.load_gather(pz_v, [ir]) - plsc.load_gather(pz_v, [ic])
      rowi = lane + j * L
      plsc.store_scatter(cbuf, [rowi, zero16], sv * dx)
      plsc.store_scatter(cbuf, [rowi, zero16 + 1], sv * dy)
      plsc.store_scatter(cbuf, [rowi, zero16 + 2], sv * dz)
    pltpu.sync_copy(cbuf, accc.at[idx_r], add=True)
    return carry

  lax.fori_loop(0, nchunk, chunk, 0)

  plsc.subcore_barrier()

  @pl.when(sid == 0)
  def _flush():
    pltpu.sync_copy(accc, aggc_hbm.at[cid])


def _sc_scatter_c(row, col, s, px, py, pz, n):
  e = row.shape[0]
  epw = e // NW
  nchunk = epw // C
  zc = jnp.zeros((n, PPAD), jnp.float32)
  mesh = plsc.VectorSubcoreMesh(core_axis_name="c", subcore_axis_name="s",
                                num_cores=NC, num_subcores=NS)
  f = pl.kernel(
      functools.partial(_sc_scatter_c_body, epw, nchunk),
      compiler_params=pltpu.CompilerParams(use_tc_tiling_on_sc=False,
                                           needs_layout_passes=False),
      out_type=jax.ShapeDtypeStruct((NC, n, PPAD), jnp.float32),
      mesh=mesh,
      scratch_types=(
          pltpu.VMEM((C,), jnp.int32),
          pltpu.VMEM((C,), jnp.int32),
          pltpu.VMEM((C,), jnp.float32),
          pltpu.VMEM((C, PPAD), jnp.float32),
          pltpu.VMEM((n,), jnp.float32),
          pltpu.VMEM((n,), jnp.float32),
          pltpu.VMEM((n,), jnp.float32),
          pltpu.VMEM_SHARED((n, PPAD), jnp.float32),
          pltpu.SemaphoreType.DMA,
      ),
  )
  return f(row, col, s, px, py, pz, zc)


def _unpack_bf16(x32):
  lo = jax.lax.bitcast_convert_type(x32 << 16, jnp.float32)
  hi = jax.lax.bitcast_convert_type(x32 & jnp.int32(-65536), jnp.float32)
  return jnp.concatenate([lo, hi], axis=1).astype(jnp.bfloat16)


def _tc_edge_body(hr, hc, w1a, w1d, b1, w2, b2, wc1, bc1, wc2,
                  m2_out, s_out):
  hrv = hr[...]
  hcv = hc[...]
  dx = (jax.lax.bitcast_convert_type(hrv[:, 64:65], jnp.float32)
        - jax.lax.bitcast_convert_type(hcv[:, 64:65], jnp.float32))
  dy = (jax.lax.bitcast_convert_type(hrv[:, 65:66], jnp.float32)
        - jax.lax.bitcast_convert_type(hcv[:, 65:66], jnp.float32))
  dz = (jax.lax.bitcast_convert_type(hrv[:, 66:67], jnp.float32)
        - jax.lax.bitcast_convert_type(hcv[:, 66:67], jnp.float32))
  d2 = dx * dx + dy * dy + dz * dz
  dist = jnp.sqrt(d2 + 1e-8)
  hb = jnp.concatenate([_unpack_bf16(hrv[:, :64]), _unpack_bf16(hcv[:, :64])],
                       axis=1)
  x = jnp.dot(hb, w1a[...], preferred_element_type=jnp.float32)
  x = (x + dist * w1d[...] + b1[...]).astype(jnp.bfloat16)
  m1 = _silu(x)
  y = jnp.dot(m1, w2[...], preferred_element_type=jnp.float32) + b2[...]
  m2 = _silu(y)
  m2b = m2.astype(jnp.bfloat16)
  z = _silu((jnp.dot(m2b, wc1[...], preferred_element_type=jnp.float32)
             + bc1[...]).astype(jnp.bfloat16)).astype(jnp.float32)
  cw = jnp.sum(z * wc2[...], axis=1, keepdims=True)
  s = cw / (dist + 1e-8)
  m2_out[...] = m2
  s_out[...] = jnp.reshape(s, (s.shape[0],))


def _tc_edge(hr, hc, w1a, w1d, b1, w2, b2, wc1, bc1, wc2):
  e = hr.shape[0]
  be = 512
  grid = (e // be,)
  blk = lambda r, c: pl.BlockSpec((r, c), lambda i: (i, 0))
  wblk = lambda r, c: pl.BlockSpec((r, c), lambda i: (0, 0))
  return pl.pallas_call(
      _tc_edge_body,
      grid=grid,
      in_specs=[
          blk(be, HID), blk(be, HID),
          wblk(2 * HID, HID), wblk(1, HID), wblk(1, HID),
          wblk(HID, HID), wblk(1, HID),
          wblk(HID, HID), wblk(1, HID), wblk(1, HID),
      ],
      out_specs=[blk(be, HID), pl.BlockSpec((be,), lambda i: (i,))],
      out_shape=[
          jax.ShapeDtypeStruct((e, HID), jnp.float32),
          jax.ShapeDtypeStruct((e,), jnp.float32),
      ],
  )(hr, hc, w1a, w1d, b1, w2, b2, wc1, bc1, wc2)


def _tc_node_body(h, posp, aggm, aggc, wn1a, wn1b, bn1, wn2, bn2, g, b,
                  h_out, posp_out):
  agg = aggm[0] + aggm[1]
  x = jnp.dot(h[...], wn1a[...], preferred_element_type=jnp.float32)
  x = x + jnp.dot(agg, wn1b[...], preferred_element_type=jnp.float32)
  x = _silu(x + bn1[...])
  hupd = jnp.dot(x, wn2[...], preferred_element_type=jnp.float32) + bn2[...]
  y = h[...] + hupd
  mu = jnp.mean(y, axis=1, keepdims=True)
  var = jnp.mean((y - mu) * (y - mu), axis=1, keepdims=True)
  h_out[...] = (y - mu) * jax.lax.rsqrt(var + 1e-5) * g[...] + b[...]
  posp_out[...] = posp[...] + aggc[0] + aggc[1]


def _tc_node(h, posp, aggm, aggc, wn1a, wn1b, bn1, wn2, bn2, g, b):
  n = h.shape[0]
  bn = 1000
  grid = (n // bn,)
  blk = lambda r, c: pl.BlockSpec((r, c), lambda i: (i, 0))
  wblk = lambda r, c: pl.BlockSpec((r, c), lambda i: (0, 0))
  blk3 = lambda r, c: pl.BlockSpec((NC, r, c), lambda i: (0, i, 0))
  return pl.pallas_call(
      _tc_node_body,
      grid=grid,
      in_specs=[
          blk(bn, HID), blk(bn, PPAD), blk3(bn, HID), blk3(bn, PPAD),
          wblk(HID, HID), wblk(HID, HID), wblk(1, HID),
          wblk(HID, HID), wblk(1, HID), wblk(1, HID), wblk(1, HID),
      ],
      out_specs=[blk(bn, HID), blk(bn, PPAD)],
      out_shape=[
          jax.ShapeDtypeStruct((n, HID), jnp.float32),
          jax.ShapeDtypeStruct((n, PPAD), jnp.float32),
      ],
  )(h, posp, aggm, aggc, wn1a, wn1b, bn1, wn2, bn2, g, b)


@jax.jit
def kernel(h, pos, edge_index, W_e1, b_e1, W_e2, b_e2, W_n1, b_n1, W_n2,
           b_n2, W_c1, b_c1, W_c2, ln_gamma, ln_beta):
  n = h.shape[0]
  row = edge_index[0].astype(jnp.int32)
  col = edge_index[1].astype(jnp.int32)
  posp = jnp.zeros((n, PPAD), jnp.float32).at[:, :3].set(pos)

  hpacked = jax.lax.bitcast_convert_type(
      h.astype(jnp.bfloat16).reshape(n, HID // 2, 2), jnp.int32)
  posbits = jax.lax.bitcast_convert_type(pos, jnp.int32)
  tbl = jnp.concatenate(
      [hpacked, posbits, jnp.zeros((n, HID - HID // 2 - 3), jnp.int32)],
      axis=1)

  hr32, hc32 = _sc_gather(tbl, row, col)

  wa = W_e1[:HID]
  wb = W_e1[HID:2 * HID]
  w1a = jnp.concatenate([wa[0::2], wa[1::2], wb[0::2], wb[1::2]],
                        axis=0).astype(jnp.bfloat16)
  w1d = W_e1[2 * HID].reshape(1, HID)
  m2, s = _tc_edge(hr32, hc32, w1a, w1d, b_e1.reshape(1, HID),
                   W_e2.astype(jnp.bfloat16), b_e2.reshape(1, HID),
                   W_c1.astype(jnp.bfloat16), b_c1.reshape(1, HID),
                   W_c2.reshape(1, HID))

  aggm = _sc_scatter_m(row, m2, n)
  aggc = _sc_scatter_c(row, col, s, pos[:, 0], pos[:, 1], pos[:, 2], n)

  h_out, posp_out = _tc_node(h, posp, aggm, aggc, W_n1[:HID], W_n1[HID:],
                             b_n1.reshape(1, HID), W_n2,
                             b_n2.reshape(1, HID), ln_gamma.reshape(1, HID),
                             ln_beta.reshape(1, HID))
  return h_out, posp_out[:, :3]


# speedup vs baseline: 2.1659x; 1.1554x over previous
import functools

import jax
import jax.numpy as jnp
from jax import lax
from jax.experimental import pallas as pl
from jax.experimental.pallas import tpu as pltpu
from jax.experimental.pallas import tpu_sc as plsc

HID = 128
PPAD = 16
NC, NS = 2, 16
NW = NC * NS
C = 80
L = 16


def _silu(x):
  one = jnp.asarray(1.0, x.dtype)
  return x / (one + jnp.exp(-x))


def _sc_gather_body(epw, nchunk, tbl_hbm, row_hbm, col_hbm,
                    hr_hbm, hc_hbm,
                    idx_r, idx_c, buf_r, buf_c, sem):
  wid = lax.axis_index("s") * NC + lax.axis_index("c")
  base = wid * epw

  def chunk(i, carry):
    off = base + i * C
    pltpu.sync_copy(row_hbm.at[pl.ds(off, C)], idx_r)
    pltpu.sync_copy(col_hbm.at[pl.ds(off, C)], idx_c)
    d1 = pltpu.async_copy(tbl_hbm.at[idx_r], buf_r, sem)
    d2 = pltpu.async_copy(tbl_hbm.at[idx_c], buf_c, sem)
    d1.wait()
    d2.wait()
    pltpu.sync_copy(buf_r, hr_hbm.at[pl.ds(off, C)])
    pltpu.sync_copy(buf_c, hc_hbm.at[pl.ds(off, C)])
    return carry

  lax.fori_loop(0, nchunk, chunk, 0)


def _sc_gather(tbl, row, col):
  e = row.shape[0]
  epw = e // NW
  nchunk = epw // C
  mesh = plsc.VectorSubcoreMesh(core_axis_name="c", subcore_axis_name="s",
                                num_cores=NC, num_subcores=NS)
  f = pl.kernel(
      functools.partial(_sc_gather_body, epw, nchunk),
      out_type=(
          jax.ShapeDtypeStruct((e, HID), jnp.int32),
          jax.ShapeDtypeStruct((e, HID), jnp.int32),
      ),
      mesh=mesh,
      scratch_types=(
          pltpu.VMEM((C,), jnp.int32),
          pltpu.VMEM((C,), jnp.int32),
          pltpu.VMEM((C, HID), jnp.int32),
          pltpu.VMEM((C, HID), jnp.int32),
          pltpu.SemaphoreType.DMA,
      ),
  )
  return f(tbl, row, col)


def _sc_scatter_m_body(epw, nchunk, row_hbm, m2_hbm, zm_hbm, aggm_hbm,
                       idx, mbuf, accm, sem):
  cid = lax.axis_index("c")
  sid = lax.axis_index("s")
  wid = sid * NC + cid

  @pl.when(sid == 0)
  def _zero():
    pltpu.sync_copy(zm_hbm, accm)

  plsc.subcore_barrier()

  base = wid * epw

  def chunk(i, carry):
    off = base + i * C
    pltpu.sync_copy(row_hbm.at[pl.ds(off, C)], idx)
    pltpu.sync_copy(m2_hbm.at[pl.ds(off, C)], mbuf)
    pltpu.sync_copy(mbuf, accm.at[idx], add=True)
    return carry

  lax.fori_loop(0, nchunk, chunk, 0)

  plsc.subcore_barrier()

  @pl.when(sid == 0)
  def _flush():
    pltpu.sync_copy(accm, aggm_hbm.at[cid])


def _sc_scatter_m(row, m2, n):
  e = row.shape[0]
  epw = e // NW
  nchunk = epw // C
  zm = jnp.zeros((n, HID), jnp.float32)
  mesh = plsc.VectorSubcoreMesh(core_axis_name="c", subcore_axis_name="s",
                                num_cores=NC, num_subcores=NS)
  f = pl.kernel(
      functools.partial(_sc_scatter_m_body, epw, nchunk),
      out_type=jax.ShapeDtypeStruct((NC, n, HID), jnp.float32),
      mesh=mesh,
      scratch_types=(
          pltpu.VMEM((C,), jnp.int32),
          pltpu.VMEM((C, HID), jnp.float32),
          pltpu.VMEM_SHARED((n, HID), jnp.float32),
          pltpu.SemaphoreType.DMA,
      ),
  )
  return f(row, m2, zm)


def _sc_scatter_c_body(epw, nchunk, row_hbm, col_hbm, s_hbm,
                       px_hbm, py_hbm, pz_hbm, zc_hbm, aggc_hbm,
                       idx_r, idx_c, sbuf, cbuf, px_v, py_v, pz_v, accc, sem):
  cid = lax.axis_index("c")
  sid = lax.axis_index("s")
  wid = sid * NC + cid

  pltpu.sync_copy(px_hbm, px_v)
  pltpu.sync_copy(py_hbm, py_v)
  pltpu.sync_copy(pz_hbm, pz_v)

  @pl.when(sid == 0)
  def _zero():
    pltpu.sync_copy(zc_hbm, accc)

  pltpu.sync_copy(zc_hbm.at[pl.ds(0, C)], cbuf)

  plsc.subcore_barrier()

  base = wid * epw
  lane = lax.iota(jnp.int32, L)
  zero16 = jnp.zeros((L,), jnp.int32)

  def chunk(i, carry):
    off = base + i * C
    pltpu.sync_copy(row_hbm.at[pl.ds(off, C)], idx_r)
    pltpu.sync_copy(col_hbm.at[pl.ds(off, C)], idx_c)
    pltpu.sync_copy(s_hbm.at[pl.ds(off, C)], sbuf)
    for j in range(C // L):
      ir = idx_r[pl.ds(j * L, L)]
      ic = idx_c[pl.ds(j * L, L)]
      sv = sbuf[pl.ds(j * L, L)]
      dx = plsc.load_gather(px_v, [ir]) - plsc.load_gather(px_v, [ic])
      dy = plsc.load_gather(py_v, [ir]) - plsc.load_gather(py_v, [ic])
      dz = plsc.load_gather(pz_v, [ir]) - plsc.load_gather(pz_v, [ic])
      rowi = lane + j * L
      plsc.store_scatter(cbuf, [rowi, zero16], sv * dx)
      plsc.store_scatter(cbuf, [rowi, zero16 + 1], sv * dy)
      plsc.store_scatter(cbuf, [rowi, zero16 + 2], sv * dz)
    pltpu.sync_copy(cbuf, accc.at[idx_r], add=True)
    return carry

  lax.fori_loop(0, nchunk, chunk, 0)

  plsc.subcore_barrier()

  @pl.when(sid == 0)
  def _flush():
    pltpu.sync_copy(accc, aggc_hbm.at[cid])


def _sc_scatter_c(row, col, s, px, py, pz, n):
  e = row.shape[0]
  epw = e // NW
  nchunk = epw // C
  zc = jnp.zeros((n, PPAD), jnp.float32)
  mesh = plsc.VectorSubcoreMesh(core_axis_name="c", subcore_axis_name="s",
                                num_cores=NC, num_subcores=NS)
  f = pl.kernel(
      functools.partial(_sc_scatter_c_body, epw, nchunk),
      compiler_params=pltpu.CompilerParams(use_tc_tiling_on_sc=False,
                                           needs_layout_passes=False),
      out_type=jax.ShapeDtypeStruct((NC, n, PPAD), jnp.float32),
      mesh=mesh,
      scratch_types=(
          pltpu.VMEM((C,), jnp.int32),
          pltpu.VMEM((C,), jnp.int32),
          pltpu.VMEM((C,), jnp.float32),
          pltpu.VMEM((C, PPAD), jnp.float32),
          pltpu.VMEM((n,), jnp.float32),
          pltpu.VMEM((n,), jnp.float32),
          pltpu.VMEM((n,), jnp.float32),
          pltpu.VMEM_SHARED((n, PPAD), jnp.float32),
          pltpu.SemaphoreType.DMA,
      ),
  )
  return f(row, col, s, px, py, pz, zc)


def _unpack_bf16(x32):
  lo = jax.lax.bitcast_convert_type(x32 << 16, jnp.float32)
  hi = jax.lax.bitcast_convert_type(x32 & jnp.int32(-65536), jnp.float32)
  return jnp.concatenate([lo, hi], axis=1).astype(jnp.bfloat16)


def _tc_edge_body(hr, hc, w1a, w1d, b1, w2, b2, wc1, bc1, wc2,
                  m2_out, s_out):
  hrv = hr[...]
  hcv = hc[...]
  dx = (jax.lax.bitcast_convert_type(hrv[:, 64:65], jnp.float32)
        - jax.lax.bitcast_convert_type(hcv[:, 64:65], jnp.float32))
  dy = (jax.lax.bitcast_convert_type(hrv[:, 65:66], jnp.float32)
        - jax.lax.bitcast_convert_type(hcv[:, 65:66], jnp.float32))
  dz = (jax.lax.bitcast_convert_type(hrv[:, 66:67], jnp.float32)
        - jax.lax.bitcast_convert_type(hcv[:, 66:67], jnp.float32))
  d2 = dx * dx + dy * dy + dz * dz
  dist = jnp.sqrt(d2 + 1e-8)
  hb = jnp.concatenate([_unpack_bf16(hrv[:, :64]), _unpack_bf16(hcv[:, :64])],
                       axis=1)
  x = jnp.dot(hb, w1a[...], preferred_element_type=jnp.float32)
  x = (x + dist * w1d[...] + b1[...]).astype(jnp.bfloat16)
  m1 = _silu(x)
  y = jnp.dot(m1, w2[...], preferred_element_type=jnp.float32) + b2[...]
  m2 = _silu(y)
  m2b = m2.astype(jnp.bfloat16)
  z = _silu((jnp.dot(m2b, wc1[...], preferred_element_type=jnp.float32)
             + bc1[...]).astype(jnp.bfloat16)).astype(jnp.float32)
  cw = jnp.sum(z * wc2[...], axis=1, keepdims=True)
  s = cw / (dist + 1e-8)
  m2_out[...] = m2
  s_out[...] = jnp.reshape(s, (s.shape[0],))


def _tc_edge(hr, hc, w1a, w1d, b1, w2, b2, wc1, bc1, wc2):
  e = hr.shape[0]
  be = 512
  grid = (e // be,)
  blk = lambda r, c: pl.BlockSpec((r, c), lambda i: (i, 0))
  wblk = lambda r, c: pl.BlockSpec((r, c), lambda i: (0, 0))
  return pl.pallas_call(
      _tc_edge_body,
      grid=grid,
      in_specs=[
          blk(be, HID), blk(be, HID),
          wblk(2 * HID, HID), wblk(1, HID), wblk(1, HID),
          wblk(HID, HID), wblk(1, HID),
          wblk(HID, HID), wblk(1, HID), wblk(1, HID),
      ],
      out_specs=[blk(be, HID), pl.BlockSpec((be,), lambda i: (i,))],
      out_shape=[
          jax.ShapeDtypeStruct((e, HID), jnp.float32),
          jax.ShapeDtypeStruct((e,), jnp.float32),
      ],
  )(hr, hc, w1a, w1d, b1, w2, b2, wc1, bc1, wc2)


def _tc_node_body(h, posp, aggm, aggc, wn1a, wn1b, bn1, wn2, bn2, g, b,
                  h_out, posp_out):
  agg = aggm[0]
  for p in range(1, aggm.shape[0]):
    agg = agg + aggm[p]
  x = jnp.dot(h[...], wn1a[...], preferred_element_type=jnp.float32)
  x = x + jnp.dot(agg, wn1b[...], preferred_element_type=jnp.float32)
  x = _silu(x + bn1[...])
  hupd = jnp.dot(x, wn2[...], preferred_element_type=jnp.float32) + bn2[...]
  y = h[...] + hupd
  mu = jnp.mean(y, axis=1, keepdims=True)
  var = jnp.mean((y - mu) * (y - mu), axis=1, keepdims=True)
  h_out[...] = (y - mu) * jax.lax.rsqrt(var + 1e-5) * g[...] + b[...]
  pout = posp[...]
  for p in range(aggc.shape[0]):
    pout = pout + aggc[p]
  posp_out[...] = pout


def _tc_node(h, posp, aggm, aggc, wn1a, wn1b, bn1, wn2, bn2, g, b):
  n = h.shape[0]
  npart = aggm.shape[0]
  bn = 1000
  grid = (n // bn,)
  blk = lambda r, c: pl.BlockSpec((r, c), lambda i: (i, 0))
  wblk = lambda r, c: pl.BlockSpec((r, c), lambda i: (0, 0))
  blk3 = lambda r, c: pl.BlockSpec((npart, r, c), lambda i: (0, i, 0))
  return pl.pallas_call(
      _tc_node_body,
      grid=grid,
      in_specs=[
          blk(bn, HID), blk(bn, PPAD), blk3(bn, HID), blk3(bn, PPAD),
          wblk(HID, HID), wblk(HID, HID), wblk(1, HID),
          wblk(HID, HID), wblk(1, HID), wblk(1, HID), wblk(1, HID),
      ],
      out_specs=[blk(bn, HID), blk(bn, PPAD)],
      out_shape=[
          jax.ShapeDtypeStruct((n, HID), jnp.float32),
          jax.ShapeDtypeStruct((n, PPAD), jnp.float32),
      ],
  )(h, posp, aggm, aggc, wn1a, wn1b, bn1, wn2, bn2, g, b)


@jax.jit
def kernel(h, pos, edge_index, W_e1, b_e1, W_e2, b_e2, W_n1, b_n1, W_n2,
           b_n2, W_c1, b_c1, W_c2, ln_gamma, ln_beta):
  n = h.shape[0]
  row = edge_index[0].astype(jnp.int32)
  col = edge_index[1].astype(jnp.int32)
  posp = jnp.zeros((n, PPAD), jnp.float32).at[:, :3].set(pos)

  hpacked = jax.lax.bitcast_convert_type(
      h.astype(jnp.bfloat16).reshape(n, HID // 2, 2), jnp.int32)
  posbits = jax.lax.bitcast_convert_type(pos, jnp.int32)
  tbl = jnp.concatenate(
      [hpacked, posbits, jnp.zeros((n, HID - HID // 2 - 3), jnp.int32)],
      axis=1)

  wa = W_e1[:HID]
  wb = W_e1[HID:2 * HID]
  w1a = jnp.concatenate([wa[0::2], wa[1::2], wb[0::2], wb[1::2]],
                        axis=0).astype(jnp.bfloat16)
  w1d = W_e1[2 * HID].reshape(1, HID)

  e = row.shape[0]
  quantum = NW * C
  nq = e // quantum
  splits = [q for q in (nq // 2,) if 0 < q < nq]
  bounds = [0] + [q * quantum for q in splits] + [e]

  aggms = []
  aggcs = []
  for lo, hi in zip(bounds[:-1], bounds[1:]):
    rk = lax.slice_in_dim(row, lo, hi)
    ck = lax.slice_in_dim(col, lo, hi)
    hr32, hc32 = _sc_gather(tbl, rk, ck)
    m2, s = _tc_edge(hr32, hc32, w1a, w1d, b_e1.reshape(1, HID),
                     W_e2.astype(jnp.bfloat16), b_e2.reshape(1, HID),
                     W_c1.astype(jnp.bfloat16), b_c1.reshape(1, HID),
                     W_c2.reshape(1, HID))
    aggms.append(_sc_scatter_m(rk, m2, n))
    aggcs.append(_sc_scatter_c(rk, ck, s, pos[:, 0], pos[:, 1], pos[:, 2], n))

  aggm = jnp.concatenate(aggms, axis=0)
  aggc = jnp.concatenate(aggcs, axis=0)

  h_out, posp_out = _tc_node(h, posp, aggm, aggc, W_n1[:HID], W_n1[HID:],
                             b_n1.reshape(1, HID), W_n2,
                             b_n2.reshape(1, HID), ln_gamma.reshape(1, HID),
                             ln_beta.reshape(1, HID))
  return h_out, posp_out[:, :3]


# speedup vs baseline: 2.2547x; 1.0410x over previous
import functools

import jax
import jax.numpy as jnp
from jax import lax
from jax.experimental import pallas as pl
from jax.experimental.pallas import tpu as pltpu
from jax.experimental.pallas import tpu_sc as plsc

HID = 128
PPAD = 16
NC, NS = 2, 16
NW = NC * NS
C = 80
L = 16


def _silu(x):
  one = jnp.asarray(1.0, x.dtype)
  return x / (one + jnp.exp(-x))


def _sc_gather_body(epw, nchunk, tbl_hbm, row_hbm, col_hbm,
                    hr_hbm, hc_hbm,
                    idx_r, idx_c, buf_r, buf_c, sem):
  wid = lax.axis_index("s") * NC + lax.axis_index("c")
  base = wid * epw

  def chunk(i, carry):
    off = base + i * C
    pltpu.sync_copy(row_hbm.at[pl.ds(off, C)], idx_r)
    pltpu.sync_copy(col_hbm.at[pl.ds(off, C)], idx_c)
    d1 = pltpu.async_copy(tbl_hbm.at[idx_r], buf_r, sem)
    d2 = pltpu.async_copy(tbl_hbm.at[idx_c], buf_c, sem)
    d1.wait()
    d2.wait()
    pltpu.sync_copy(buf_r, hr_hbm.at[pl.ds(off, C)])
    pltpu.sync_copy(buf_c, hc_hbm.at[pl.ds(off, C)])
    return carry

  lax.fori_loop(0, nchunk, chunk, 0)


def _sc_gather(tbl, row, col):
  e = row.shape[0]
  epw = e // NW
  nchunk = epw // C
  mesh = plsc.VectorSubcoreMesh(core_axis_name="c", subcore_axis_name="s",
                                num_cores=NC, num_subcores=NS)
  f = pl.kernel(
      functools.partial(_sc_gather_body, epw, nchunk),
      out_type=(
          jax.ShapeDtypeStruct((e, HID), jnp.int32),
          jax.ShapeDtypeStruct((e, HID), jnp.int32),
      ),
      mesh=mesh,
      scratch_types=(
          pltpu.VMEM((C,), jnp.int32),
          pltpu.VMEM((C,), jnp.int32),
          pltpu.VMEM((C, HID), jnp.int32),
          pltpu.VMEM((C, HID), jnp.int32),
          pltpu.SemaphoreType.DMA,
      ),
  )
  return f(tbl, row, col)


def _sc_scatter_m_body(epw, nchunk, row_hbm, m2_hbm, zm_hbm, aggm_hbm,
                       idx, mbuf, accm, sem):
  cid = lax.axis_index("c")
  sid = lax.axis_index("s")
  wid = sid * NC + cid

  @pl.when(sid == 0)
  def _zero():
    pltpu.sync_copy(zm_hbm, accm)

  plsc.subcore_barrier()

  base = wid * epw

  def chunk(i, carry):
    off = base + i * C
    pltpu.sync_copy(row_hbm.at[pl.ds(off, C)], idx)
    pltpu.sync_copy(m2_hbm.at[pl.ds(off, C)], mbuf)
    pltpu.sync_copy(mbuf, accm.at[idx], add=True)
    return carry

  lax.fori_loop(0, nchunk, chunk, 0)

  plsc.subcore_barrier()

  @pl.when(sid == 0)
  def _flush():
    pltpu.sync_copy(accm, aggm_hbm.at[cid])


def _sc_scatter_m(row, m2, n):
  e = row.shape[0]
  epw = e // NW
  nchunk = epw // C
  zm = jnp.zeros((n, HID), jnp.float32)
  mesh = plsc.VectorSubcoreMesh(core_axis_name="c", subcore_axis_name="s",
                                num_cores=NC, num_subcores=NS)
  f = pl.kernel(
      functools.partial(_sc_scatter_m_body, epw, nchunk),
      out_type=jax.ShapeDtypeStruct((NC, n, HID), jnp.float32),
      mesh=mesh,
      scratch_types=(
          pltpu.VMEM((C,), jnp.int32),
          pltpu.VMEM((C, HID), jnp.float32),
          pltpu.VMEM_SHARED((n, HID), jnp.float32),
          pltpu.SemaphoreType.DMA,
      ),
  )
  return f(row, m2, zm)


def _sc_scatter_c_body(epw, nchunk, row_hbm, col_hbm, s_hbm,
                       px_hbm, py_hbm, pz_hbm, zc_hbm, aggc_hbm,
                       idx_r, idx_c, sbuf, cbuf, px_v, py_v, pz_v, accc, sem):
  cid = lax.axis_index("c")
  sid = lax.axis_index("s")
  wid = sid * NC + cid

  pltpu.sync_copy(px_hbm, px_v)
  pltpu.sync_copy(py_hbm, py_v)
  pltpu.sync_copy(pz_hbm, pz_v)

  @pl.when(sid == 0)
  def _zero():
    pltpu.sync_copy(zc_hbm, accc)

  pltpu.sync_copy(zc_hbm.at[pl.ds(0, C)], cbuf)

  plsc.subcore_barrier()

  base = wid * epw
  lane = lax.iota(jnp.int32, L)
  zero16 = jnp.zeros((L,), jnp.int32)

  def chunk(i, carry):
    off = base + i * C
    pltpu.sync_copy(row_hbm.at[pl.ds(off, C)], idx_r)
    pltpu.sync_copy(col_hbm.at[pl.ds(off, C)], idx_c)
    pltpu.sync_copy(s_hbm.at[pl.ds(off, C)], sbuf)
    for j in range(C // L):
      ir = idx_r[pl.ds(j * L, L)]
      ic = idx_c[pl.ds(j * L, L)]
      sv = sbuf[pl.ds(j * L, L)]
      dx = plsc.load_gather(px_v, [ir]) - plsc.load_gather(px_v, [ic])
      dy = plsc.load_gather(py_v, [ir]) - plsc.load_gather(py_v, [ic])
      dz = plsc.load_gather(pz_v, [ir]) - plsc.load_gather(pz_v, [ic])
      rowi = lane + j * L
      plsc.store_scatter(cbuf, [rowi, zero16], sv * dx)
      plsc.store_scatter(cbuf, [rowi, zero16 + 1], sv * dy)
      plsc.store_scatter(cbuf, [rowi, zero16 + 2], sv * dz)
    pltpu.sync_copy(cbuf, accc.at[idx_r], add=True)
    return carry

  lax.fori_loop(0, nchunk, chunk, 0)

  plsc.subcore_barrier()

  @pl.when(sid == 0)
  def _flush():
    pltpu.sync_copy(accc, aggc_hbm.at[cid])


def _sc_scatter_c(row, col, s, px, py, pz, n):
  e = row.shape[0]
  epw = e // NW
  nchunk = epw // C
  zc = jnp.zeros((n, PPAD), jnp.float32)
  mesh = plsc.VectorSubcoreMesh(core_axis_name="c", subcore_axis_name="s",
                                num_cores=NC, num_subcores=NS)
  f = pl.kernel(
      functools.partial(_sc_scatter_c_body, epw, nchunk),
      compiler_params=pltpu.CompilerParams(use_tc_tiling_on_sc=False,
                                           needs_layout_passes=False),
      out_type=jax.ShapeDtypeStruct((NC, n, PPAD), jnp.float32),
      mesh=mesh,
      scratch_types=(
          pltpu.VMEM((C,), jnp.int32),
          pltpu.VMEM((C,), jnp.int32),
          pltpu.VMEM((C,), jnp.float32),
          pltpu.VMEM((C, PPAD), jnp.float32),
          pltpu.VMEM((n,), jnp.float32),
          pltpu.VMEM((n,), jnp.float32),
          pltpu.VMEM((n,), jnp.float32),
          pltpu.VMEM_SHARED((n, PPAD), jnp.float32),
          pltpu.SemaphoreType.DMA,
      ),
  )
  return f(row, col, s, px, py, pz, zc)


def _unpack_bf16(x32):
  lo = jax.lax.bitcast_convert_type(x32 << 16, jnp.float32)
  hi = jax.lax.bitcast_convert_type(x32 & jnp.int32(-65536), jnp.float32)
  return jnp.concatenate([lo, hi], axis=1).astype(jnp.bfloat16)


def _tc_edge_body(hr, hc, w1a, w1d, b1, w2, b2, wc1, bc1, wc2,
                  m2_out, s_out):
  hrv = hr[...]
  hcv = hc[...]
  dx = (jax.lax.bitcast_convert_type(hrv[:, 64:65], jnp.float32)
        - jax.lax.bitcast_convert_type(hcv[:, 64:65], jnp.float32))
  dy = (jax.lax.bitcast_convert_type(hrv[:, 65:66], jnp.float32)
        - jax.lax.bitcast_convert_type(hcv[:, 65:66], jnp.float32))
  dz = (jax.lax.bitcast_convert_type(hrv[:, 66:67], jnp.float32)
        - jax.lax.bitcast_convert_type(hcv[:, 66:67], jnp.float32))
  d2 = dx * dx + dy * dy + dz * dz
  dist = jnp.sqrt(d2 + 1e-8)
  hb = jnp.concatenate([_unpack_bf16(hrv[:, :64]), _unpack_bf16(hcv[:, :64])],
                       axis=1)
  x = jnp.dot(hb, w1a[...], preferred_element_type=jnp.float32)
  x = (x + dist * w1d[...] + b1[...]).astype(jnp.bfloat16)
  m1 = _silu(x)
  y = jnp.dot(m1, w2[...], preferred_element_type=jnp.float32) + b2[...]
  m2 = _silu(y)
  m2b = m2.astype(jnp.bfloat16)
  z = _silu((jnp.dot(m2b, wc1[...], preferred_element_type=jnp.float32)
             + bc1[...]).astype(jnp.bfloat16)).astype(jnp.float32)
  cw = jnp.sum(z * wc2[...], axis=1, keepdims=True)
  s = cw / (dist + 1e-8)
  m2_out[...] = m2
  s_out[...] = jnp.reshape(s, (s.shape[0],))


def _tc_edge(hr, hc, w1a, w1d, b1, w2, b2, wc1, bc1, wc2):
  e = hr.shape[0]
  be = 512
  grid = (e // be,)
  blk = lambda r, c: pl.BlockSpec((r, c), lambda i: (i, 0))
  wblk = lambda r, c: pl.BlockSpec((r, c), lambda i: (0, 0))
  return pl.pallas_call(
      _tc_edge_body,
      grid=grid,
      in_specs=[
          blk(be, HID), blk(be, HID),
          wblk(2 * HID, HID), wblk(1, HID), wblk(1, HID),
          wblk(HID, HID), wblk(1, HID),
          wblk(HID, HID), wblk(1, HID), wblk(1, HID),
      ],
      out_specs=[blk(be, HID), pl.BlockSpec((be,), lambda i: (i,))],
      out_shape=[
          jax.ShapeDtypeStruct((e, HID), jnp.float32),
          jax.ShapeDtypeStruct((e,), jnp.float32),
      ],
  )(hr, hc, w1a, w1d, b1, w2, b2, wc1, bc1, wc2)


def _tc_node_body(h, posp, aggm, aggc, wn1a, wn1b, bn1, wn2, bn2, g, b,
                  h_out, posp_out):
  agg = aggm[0]
  for p in range(1, aggm.shape[0]):
    agg = agg + aggm[p]
  x = jnp.dot(h[...], wn1a[...], preferred_element_type=jnp.float32)
  x = x + jnp.dot(agg, wn1b[...], preferred_element_type=jnp.float32)
  x = _silu(x + bn1[...])
  hupd = jnp.dot(x, wn2[...], preferred_element_type=jnp.float32) + bn2[...]
  y = h[...] + hupd
  mu = jnp.mean(y, axis=1, keepdims=True)
  var = jnp.mean((y - mu) * (y - mu), axis=1, keepdims=True)
  h_out[...] = (y - mu) * jax.lax.rsqrt(var + 1e-5) * g[...] + b[...]
  pout = posp[...]
  for p in range(aggc.shape[0]):
    pout = pout + aggc[p]
  posp_out[...] = pout


def _tc_node(h, posp, aggm, aggc, wn1a, wn1b, bn1, wn2, bn2, g, b):
  n = h.shape[0]
  npart = aggm.shape[0]
  bn = 1000
  grid = (n // bn,)
  blk = lambda r, c: pl.BlockSpec((r, c), lambda i: (i, 0))
  wblk = lambda r, c: pl.BlockSpec((r, c), lambda i: (0, 0))
  blk3 = lambda r, c: pl.BlockSpec((npart, r, c), lambda i: (0, i, 0))
  return pl.pallas_call(
      _tc_node_body,
      grid=grid,
      in_specs=[
          blk(bn, HID), blk(bn, PPAD), blk3(bn, HID), blk3(bn, PPAD),
          wblk(HID, HID), wblk(HID, HID), wblk(1, HID),
          wblk(HID, HID), wblk(1, HID), wblk(1, HID), wblk(1, HID),
      ],
      out_specs=[blk(bn, HID), blk(bn, PPAD)],
      out_shape=[
          jax.ShapeDtypeStruct((n, HID), jnp.float32),
          jax.ShapeDtypeStruct((n, PPAD), jnp.float32),
      ],
  )(h, posp, aggm, aggc, wn1a, wn1b, bn1, wn2, bn2, g, b)


@jax.jit
def kernel(h, pos, edge_index, W_e1, b_e1, W_e2, b_e2, W_n1, b_n1, W_n2,
           b_n2, W_c1, b_c1, W_c2, ln_gamma, ln_beta):
  n = h.shape[0]
  row = edge_index[0].astype(jnp.int32)
  col = edge_index[1].astype(jnp.int32)
  posp = jnp.zeros((n, PPAD), jnp.float32).at[:, :3].set(pos)

  hpacked = jax.lax.bitcast_convert_type(
      h.astype(jnp.bfloat16).reshape(n, HID // 2, 2), jnp.int32)
  posbits = jax.lax.bitcast_convert_type(pos, jnp.int32)
  tbl = jnp.concatenate(
      [hpacked, posbits, jnp.zeros((n, HID - HID // 2 - 3), jnp.int32)],
      axis=1)

  wa = W_e1[:HID]
  wb = W_e1[HID:2 * HID]
  w1a = jnp.concatenate([wa[0::2], wa[1::2], wb[0::2], wb[1::2]],
                        axis=0).astype(jnp.bfloat16)
  w1d = W_e1[2 * HID].reshape(1, HID)

  e = row.shape[0]
  quantum = NW * C
  nq = e // quantum
  splits = [q for q in (nq // 4, nq // 2, 3 * nq // 4) if 0 < q < nq]
  bounds = [0] + [q * quantum for q in splits] + [e]

  aggms = []
  aggcs = []
  for lo, hi in zip(bounds[:-1], bounds[1:]):
    rk = lax.slice_in_dim(row, lo, hi)
    ck = lax.slice_in_dim(col, lo, hi)
    hr32, hc32 = _sc_gather(tbl, rk, ck)
    m2, s = _tc_edge(hr32, hc32, w1a, w1d, b_e1.reshape(1, HID),
                     W_e2.astype(jnp.bfloat16), b_e2.reshape(1, HID),
                     W_c1.astype(jnp.bfloat16), b_c1.reshape(1, HID),
                     W_c2.reshape(1, HID))
    aggms.append(_sc_scatter_m(rk, m2, n))
    aggcs.append(_sc_scatter_c(rk, ck, s, pos[:, 0], pos[:, 1], pos[:, 2], n))

  aggm = jnp.concatenate(aggms, axis=0)
  aggc = jnp.concatenate(aggcs, axis=0)

  h_out, posp_out = _tc_node(h, posp, aggm, aggc, W_n1[:HID], W_n1[HID:],
                             b_n1.reshape(1, HID), W_n2,
                             b_n2.reshape(1, HID), ln_gamma.reshape(1, HID),
                             ln_beta.reshape(1, HID))
  return h_out, posp_out[:, :3]


# speedup vs baseline: 2.3161x; 1.0272x over previous
import functools

import jax
import jax.numpy as jnp
from jax import lax
from jax.experimental import pallas as pl
from jax.experimental.pallas import tpu as pltpu
from jax.experimental.pallas import tpu_sc as plsc

HID = 128
PPAD = 16
NC, NS = 2, 16
NW = NC * NS
C = 80
L = 16


def _silu(x):
  one = jnp.asarray(1.0, x.dtype)
  return x / (one + jnp.exp(-x))


def _sc_gather_body(epw, nchunk, tbl_hbm, row_hbm, col_hbm,
                    hr_hbm, hc_hbm,
                    idx_r, idx_c, buf_r, buf_c, sem):
  wid = lax.axis_index("s") * NC + lax.axis_index("c")
  base = wid * epw

  def chunk(i, carry):
    off = base + i * C
    pltpu.sync_copy(row_hbm.at[pl.ds(off, C)], idx_r)
    pltpu.sync_copy(col_hbm.at[pl.ds(off, C)], idx_c)
    d1 = pltpu.async_copy(tbl_hbm.at[idx_r], buf_r, sem)
    d2 = pltpu.async_copy(tbl_hbm.at[idx_c], buf_c, sem)
    d1.wait()
    d2.wait()
    pltpu.sync_copy(buf_r, hr_hbm.at[pl.ds(off, C)])
    pltpu.sync_copy(buf_c, hc_hbm.at[pl.ds(off, C)])
    return carry

  lax.fori_loop(0, nchunk, chunk, 0)


def _sc_gather(tbl, row, col):
  e = row.shape[0]
  epw = e // NW
  nchunk = epw // C
  mesh = plsc.VectorSubcoreMesh(core_axis_name="c", subcore_axis_name="s",
                                num_cores=NC, num_subcores=NS)
  f = pl.kernel(
      functools.partial(_sc_gather_body, epw, nchunk),
      out_type=(
          jax.ShapeDtypeStruct((e, HID), jnp.int32),
          jax.ShapeDtypeStruct((e, HID), jnp.int32),
      ),
      mesh=mesh,
      scratch_types=(
          pltpu.VMEM((C,), jnp.int32),
          pltpu.VMEM((C,), jnp.int32),
          pltpu.VMEM((C, HID), jnp.int32),
          pltpu.VMEM((C, HID), jnp.int32),
          pltpu.SemaphoreType.DMA,
      ),
  )
  return f(tbl, row, col)


def _sc_scatter_m_body(epw, nchunk, row_hbm, m2_hbm, zm_hbm, aggm_hbm,
                       idx, mbuf, accm, sem):
  cid = lax.axis_index("c")
  sid = lax.axis_index("s")
  wid = sid * NC + cid

  @pl.when(sid == 0)
  def _zero():
    pltpu.sync_copy(zm_hbm, accm)

  plsc.subcore_barrier()

  base = wid * epw

  def chunk(i, carry):
    off = base + i * C
    pltpu.sync_copy(row_hbm.at[pl.ds(off, C)], idx)
    pltpu.sync_copy(m2_hbm.at[pl.ds(off, C)], mbuf)
    pltpu.sync_copy(mbuf, accm.at[idx], add=True)
    return carry

  lax.fori_loop(0, nchunk, chunk, 0)

  plsc.subcore_barrier()

  @pl.when(sid == 0)
  def _flush():
    pltpu.sync_copy(accm, aggm_hbm.at[cid])


def _sc_scatter_m(row, m2, n):
  e = row.shape[0]
  epw = e // NW
  nchunk = epw // C
  zm = jnp.zeros((n, HID), jnp.float32)
  mesh = plsc.VectorSubcoreMesh(core_axis_name="c", subcore_axis_name="s",
                                num_cores=NC, num_subcores=NS)
  f = pl.kernel(
      functools.partial(_sc_scatter_m_body, epw, nchunk),
      out_type=jax.ShapeDtypeStruct((NC, n, HID), jnp.float32),
      mesh=mesh,
      scratch_types=(
          pltpu.VMEM((C,), jnp.int32),
          pltpu.VMEM((C, HID), jnp.float32),
          pltpu.VMEM_SHARED((n, HID), jnp.float32),
          pltpu.SemaphoreType.DMA,
      ),
  )
  return f(row, m2, zm)


def _sc_scatter_c_body(epw, nchunk, row2_hbm, col_hbm, s_hbm,
                       px_hbm, py_hbm, pz_hbm, zc_hbm, aggc_hbm,
                       idx_r2, colbuf, sbuf, cbuf, px_v, py_v, pz_v, accc,
                       sem):
  cid = lax.axis_index("c")
  sid = lax.axis_index("s")
  wid = sid * NC + cid

  pltpu.sync_copy(px_hbm, px_v)
  pltpu.sync_copy(py_hbm, py_v)
  pltpu.sync_copy(pz_hbm, pz_v)

  @pl.when(sid == 0)
  def _zero():
    pltpu.sync_copy(zc_hbm, accc)

  pltpu.sync_copy(zc_hbm.at[pl.ds(0, C)], cbuf)

  pltpu.sync_copy(row2_hbm.at[pl.ds(wid * nchunk, nchunk)], idx_r2)
  pltpu.sync_copy(col_hbm.at[pl.ds(wid * epw, epw)], colbuf)
  pltpu.sync_copy(s_hbm.at[pl.ds(wid * epw, epw)], sbuf)

  plsc.subcore_barrier()

  lane = lax.iota(jnp.int32, L)
  zero16 = jnp.zeros((L,), jnp.int32)

  def chunk(i, carry):
    for j in range(C // L):
      ir = idx_r2[i, pl.ds(j * L, L)]
      ic = colbuf[pl.ds(i * C + j * L, L)]
      sv = sbuf[pl.ds(i * C + j * L, L)]
      dx = plsc.load_gather(px_v, [ir]) - plsc.load_gather(px_v, [ic])
      dy = plsc.load_gather(py_v, [ir]) - plsc.load_gather(py_v, [ic])
      dz = plsc.load_gather(pz_v, [ir]) - plsc.load_gather(pz_v, [ic])
      rowi = lane + j * L
      plsc.store_scatter(cbuf, [rowi, zero16], sv * dx)
      plsc.store_scatter(cbuf, [rowi, zero16 + 1], sv * dy)
      plsc.store_scatter(cbuf, [rowi, zero16 + 2], sv * dz)
    pltpu.sync_copy(cbuf, accc.at[idx_r2.at[i]], add=True)
    return carry

  lax.fori_loop(0, nchunk, chunk, 0)

  plsc.subcore_barrier()

  @pl.when(sid == 0)
  def _flush():
    pltpu.sync_copy(accc, aggc_hbm.at[cid])


def _sc_scatter_c(row, col, s, px, py, pz, n):
  e = row.shape[0]
  epw = e // NW
  nchunk = epw // C
  row2 = row.reshape(e // C, C)
  zc = jnp.zeros((n, PPAD), jnp.float32)
  mesh = plsc.VectorSubcoreMesh(core_axis_name="c", subcore_axis_name="s",
                                num_cores=NC, num_subcores=NS)
  f = pl.kernel(
      functools.partial(_sc_scatter_c_body, epw, nchunk),
      compiler_params=pltpu.CompilerParams(use_tc_tiling_on_sc=False,
                                           needs_layout_passes=False),
      out_type=jax.ShapeDtypeStruct((NC, n, PPAD), jnp.float32),
      mesh=mesh,
      scratch_types=(
          pltpu.VMEM((nchunk, C), jnp.int32),
          pltpu.VMEM((epw,), jnp.int32),
          pltpu.VMEM((epw,), jnp.float32),
          pltpu.VMEM((C, PPAD), jnp.float32),
          pltpu.VMEM((n,), jnp.float32),
          pltpu.VMEM((n,), jnp.float32),
          pltpu.VMEM((n,), jnp.float32),
          pltpu.VMEM_SHARED((n, PPAD), jnp.float32),
          pltpu.SemaphoreType.DMA,
      ),
  )
  return f(row2, col, s, px, py, pz, zc)


def _unpack_bf16(x32):
  lo = jax.lax.bitcast_convert_type(x32 << 16, jnp.float32)
  hi = jax.lax.bitcast_convert_type(x32 & jnp.int32(-65536), jnp.float32)
  return jnp.concatenate([lo, hi], axis=1).astype(jnp.bfloat16)


def _tc_edge_body(hr, hc, w1a, w1d, b1, w2, b2, wc1, bc1, wc2,
                  m2_out, s_out):
  hrv = hr[...]
  hcv = hc[...]
  dx = (jax.lax.bitcast_convert_type(hrv[:, 64:65], jnp.float32)
        - jax.lax.bitcast_convert_type(hcv[:, 64:65], jnp.float32))
  dy = (jax.lax.bitcast_convert_type(hrv[:, 65:66], jnp.float32)
        - jax.lax.bitcast_convert_type(hcv[:, 65:66], jnp.float32))
  dz = (jax.lax.bitcast_convert_type(hrv[:, 66:67], jnp.float32)
        - jax.lax.bitcast_convert_type(hcv[:, 66:67], jnp.float32))
  d2 = dx * dx + dy * dy + dz * dz
  dist = jnp.sqrt(d2 + 1e-8)
  hb = jnp.concatenate([_unpack_bf16(hrv[:, :64]), _unpack_bf16(hcv[:, :64])],
                       axis=1)
  x = jnp.dot(hb, w1a[...], preferred_element_type=jnp.float32)
  x = (x + dist * w1d[...] + b1[...]).astype(jnp.bfloat16)
  m1 = _silu(x)
  y = jnp.dot(m1, w2[...], preferred_element_type=jnp.float32) + b2[...]
  m2 = _silu(y)
  m2b = m2.astype(jnp.bfloat16)
  z = _silu((jnp.dot(m2b, wc1[...], preferred_element_type=jnp.float32)
             + bc1[...]).astype(jnp.bfloat16)).astype(jnp.float32)
  cw = jnp.sum(z * wc2[...], axis=1, keepdims=True)
  s = cw / (dist + 1e-8)
  m2_out[...] = m2
  s_out[...] = jnp.reshape(s, (s.shape[0],))


def _tc_edge(hr, hc, w1a, w1d, b1, w2, b2, wc1, bc1, wc2):
  e = hr.shape[0]
  be = 512
  grid = (e // be,)
  blk = lambda r, c: pl.BlockSpec((r, c), lambda i: (i, 0))
  wblk = lambda r, c: pl.BlockSpec((r, c), lambda i: (0, 0))
  return pl.pallas_call(
      _tc_edge_body,
      grid=grid,
      in_specs=[
          blk(be, HID), blk(be, HID),
          wblk(2 * HID, HID), wblk(1, HID), wblk(1, HID),
          wblk(HID, HID), wblk(1, HID),
          wblk(HID, HID), wblk(1, HID), wblk(1, HID),
      ],
      out_specs=[blk(be, HID), pl.BlockSpec((be,), lambda i: (i,))],
      out_shape=[
          jax.ShapeDtypeStruct((e, HID), jnp.float32),
          jax.ShapeDtypeStruct((e,), jnp.float32),
      ],
  )(hr, hc, w1a, w1d, b1, w2, b2, wc1, bc1, wc2)


def _tc_node_body(h, posp, aggm, aggc, wn1a, wn1b, bn1, wn2, bn2, g, b,
                  h_out, posp_out):
  agg = aggm[0]
  for p in range(1, aggm.shape[0]):
    agg = agg + aggm[p]
  x = jnp.dot(h[...], wn1a[...], preferred_element_type=jnp.float32)
  x = x + jnp.dot(agg, wn1b[...], preferred_element_type=jnp.float32)
  x = _silu(x + bn1[...])
  hupd = jnp.dot(x, wn2[...], preferred_element_type=jnp.float32) + bn2[...]
  y = h[...] + hupd
  mu = jnp.mean(y, axis=1, keepdims=True)
  var = jnp.mean((y - mu) * (y - mu), axis=1, keepdims=True)
  h_out[...] = (y - mu) * jax.lax.rsqrt(var + 1e-5) * g[...] + b[...]
  pout = posp[...]
  for p in range(aggc.shape[0]):
    pout = pout + aggc[p]
  posp_out[...] = pout


def _tc_node(h, posp, aggm, aggc, wn1a, wn1b, bn1, wn2, bn2, g, b):
  n = h.shape[0]
  npart = aggm.shape[0]
  bn = 1000
  grid = (n // bn,)
  blk = lambda r, c: pl.BlockSpec((r, c), lambda i: (i, 0))
  wblk = lambda r, c: pl.BlockSpec((r, c), lambda i: (0, 0))
  blk3 = lambda r, c: pl.BlockSpec((npart, r, c), lambda i: (0, i, 0))
  return pl.pallas_call(
      _tc_node_body,
      grid=grid,
      in_specs=[
          blk(bn, HID), blk(bn, PPAD), blk3(bn, HID), blk3(bn, PPAD),
          wblk(HID, HID), wblk(HID, HID), wblk(1, HID),
          wblk(HID, HID), wblk(1, HID), wblk(1, HID), wblk(1, HID),
      ],
      out_specs=[blk(bn, HID), blk(bn, PPAD)],
      out_shape=[
          jax.ShapeDtypeStruct((n, HID), jnp.float32),
          jax.ShapeDtypeStruct((n, PPAD), jnp.float32),
      ],
  )(h, posp, aggm, aggc, wn1a, wn1b, bn1, wn2, bn2, g, b)


@jax.jit
def kernel(h, pos, edge_index, W_e1, b_e1, W_e2, b_e2, W_n1, b_n1, W_n2,
           b_n2, W_c1, b_c1, W_c2, ln_gamma, ln_beta):
  n = h.shape[0]
  row = edge_index[0].astype(jnp.int32)
  col = edge_index[1].astype(jnp.int32)
  posp = jnp.zeros((n, PPAD), jnp.float32).at[:, :3].set(pos)

  hpacked = jax.lax.bitcast_convert_type(
      h.astype(jnp.bfloat16).reshape(n, HID // 2, 2), jnp.int32)
  posbits = jax.lax.bitcast_convert_type(pos, jnp.int32)
  tbl = jnp.concatenate(
      [hpacked, posbits, jnp.zeros((n, HID - HID // 2 - 3), jnp.int32)],
      axis=1)

  wa = W_e1[:HID]
  wb = W_e1[HID:2 * HID]
  w1a = jnp.concatenate([wa[0::2], wa[1::2], wb[0::2], wb[1::2]],
                        axis=0).astype(jnp.bfloat16)
  w1d = W_e1[2 * HID].reshape(1, HID)

  e = row.shape[0]
  quantum = NW * C
  nq = e // quantum
  splits = [q for q in (nq // 4, nq // 2, 3 * nq // 4) if 0 < q < nq]
  bounds = [0] + [q * quantum for q in splits] + [e]

  aggms = []
  aggcs = []
  for lo, hi in zip(bounds[:-1], bounds[1:]):
    rk = lax.slice_in_dim(row, lo, hi)
    ck = lax.slice_in_dim(col, lo, hi)
    hr32, hc32 = _sc_gather(tbl, rk, ck)
    m2, s = _tc_edge(hr32, hc32, w1a, w1d, b_e1.reshape(1, HID),
                     W_e2.astype(jnp.bfloat16), b_e2.reshape(1, HID),
                     W_c1.astype(jnp.bfloat16), b_c1.reshape(1, HID),
                     W_c2.reshape(1, HID))
    aggms.append(_sc_scatter_m(rk, m2, n))
    aggcs.append(_sc_scatter_c(rk, ck, s, pos[:, 0], pos[:, 1], pos[:, 2], n))

  aggm = jnp.concatenate(aggms, axis=0)
  aggc = jnp.concatenate(aggcs, axis=0)

  h_out, posp_out = _tc_node(h, posp, aggm, aggc, W_n1[:HID], W_n1[HID:],
                             b_n1.reshape(1, HID), W_n2,
                             b_n2.reshape(1, HID), ln_gamma.reshape(1, HID),
                             ln_beta.reshape(1, HID))
  return h_out, posp_out[:, :3]


# speedup vs baseline: 2.3507x; 1.0150x over previous
import functools

import jax
import jax.numpy as jnp
from jax import lax
from jax.experimental import pallas as pl
from jax.experimental.pallas import tpu as pltpu
from jax.experimental.pallas import tpu_sc as plsc

HID = 128
PPAD = 16
NC, NS = 2, 16
NW = NC * NS
C = 80
L = 16


def _silu(x):
  one = jnp.asarray(1.0, x.dtype)
  return x / (one + jnp.exp(-x))


def _sc_gather_body(epw, nchunk, tbl_hbm, row_hbm, col_hbm,
                    hr_hbm, hc_hbm,
                    idx_r, idx_c, buf_r, buf_c, sem):
  wid = lax.axis_index("s") * NC + lax.axis_index("c")
  base = wid * epw

  pltpu.sync_copy(row_hbm.at[pl.ds(base, epw)], idx_r)
  pltpu.sync_copy(col_hbm.at[pl.ds(base, epw)], idx_c)

  def chunk(i, carry):
    off = base + i * C
    d1 = pltpu.async_copy(tbl_hbm.at[idx_r.at[pl.ds(i * C, C)]], buf_r, sem)
    d2 = pltpu.async_copy(tbl_hbm.at[idx_c.at[pl.ds(i * C, C)]], buf_c, sem)
    d1.wait()
    d2.wait()
    pltpu.sync_copy(buf_r, hr_hbm.at[pl.ds(off, C)])
    pltpu.sync_copy(buf_c, hc_hbm.at[pl.ds(off, C)])
    return carry

  lax.fori_loop(0, nchunk, chunk, 0)


def _sc_gather(tbl, row, col):
  e = row.shape[0]
  epw = e // NW
  nchunk = epw // C
  mesh = plsc.VectorSubcoreMesh(core_axis_name="c", subcore_axis_name="s",
                                num_cores=NC, num_subcores=NS)
  f = pl.kernel(
      functools.partial(_sc_gather_body, epw, nchunk),
      out_type=(
          jax.ShapeDtypeStruct((e, HID), jnp.int32),
          jax.ShapeDtypeStruct((e, HID), jnp.int32),
      ),
      mesh=mesh,
      scratch_types=(
          pltpu.VMEM((epw,), jnp.int32),
          pltpu.VMEM((epw,), jnp.int32),
          pltpu.VMEM((C, HID), jnp.int32),
          pltpu.VMEM((C, HID), jnp.int32),
          pltpu.SemaphoreType.DMA,
      ),
  )
  return f(tbl, row, col)


def _sc_scatter_m_body(epw, nchunk, row_hbm, m2_hbm, zm_hbm, aggm_hbm,
                       idx, mbuf, accm, sem):
  cid = lax.axis_index("c")
  sid = lax.axis_index("s")
  wid = sid * NC + cid

  @pl.when(sid == 0)
  def _zero():
    pltpu.sync_copy(zm_hbm, accm)

  plsc.subcore_barrier()

  base = wid * epw

  def chunk(i, carry):
    off = base + i * C
    pltpu.sync_copy(row_hbm.at[pl.ds(off, C)], idx)
    pltpu.sync_copy(m2_hbm.at[pl.ds(off, C)], mbuf)
    pltpu.sync_copy(mbuf, accm.at[idx], add=True)
    return carry

  lax.fori_loop(0, nchunk, chunk, 0)

  plsc.subcore_barrier()

  @pl.when(sid == 0)
  def _flush():
    pltpu.sync_copy(accm, aggm_hbm.at[cid])


def _sc_scatter_m(row, m2, n):
  e = row.shape[0]
  epw = e // NW
  nchunk = epw // C
  zm = jnp.zeros((n, HID), jnp.float32)
  mesh = plsc.VectorSubcoreMesh(core_axis_name="c", subcore_axis_name="s",
                                num_cores=NC, num_subcores=NS)
  f = pl.kernel(
      functools.partial(_sc_scatter_m_body, epw, nchunk),
      out_type=jax.ShapeDtypeStruct((NC, n, HID), jnp.float32),
      mesh=mesh,
      scratch_types=(
          pltpu.VMEM((C,), jnp.int32),
          pltpu.VMEM((C, HID), jnp.float32),
          pltpu.VMEM_SHARED((n, HID), jnp.float32),
          pltpu.SemaphoreType.DMA,
      ),
  )
  return f(row, m2, zm)


def _sc_scatter_c_body(epw, nchunk, row2_hbm, col_hbm, s_hbm,
                       px_hbm, py_hbm, pz_hbm, zc_hbm, aggc_hbm,
                       idx_r2, colbuf, sbuf, cbuf, px_v, py_v, pz_v, accc,
                       sem):
  cid = lax.axis_index("c")
  sid = lax.axis_index("s")
  wid = sid * NC + cid

  pltpu.sync_copy(px_hbm, px_v)
  pltpu.sync_copy(py_hbm, py_v)
  pltpu.sync_copy(pz_hbm, pz_v)

  @pl.when(sid == 0)
  def _zero():
    pltpu.sync_copy(zc_hbm, accc)

  pltpu.sync_copy(zc_hbm.at[pl.ds(0, C)], cbuf)

  pltpu.sync_copy(row2_hbm.at[pl.ds(wid * nchunk, nchunk)], idx_r2)
  pltpu.sync_copy(col_hbm.at[pl.ds(wid * epw, epw)], colbuf)
  pltpu.sync_copy(s_hbm.at[pl.ds(wid * epw, epw)], sbuf)

  plsc.subcore_barrier()

  lane = lax.iota(jnp.int32, L)
  zero16 = jnp.zeros((L,), jnp.int32)

  def chunk(i, carry):
    for j in range(C // L):
      ir = idx_r2[i, pl.ds(j * L, L)]
      ic = colbuf[pl.ds(i * C + j * L, L)]
      sv = sbuf[pl.ds(i * C + j * L, L)]
      dx = plsc.load_gather(px_v, [ir]) - plsc.load_gather(px_v, [ic])
      dy = plsc.load_gather(py_v, [ir]) - plsc.load_gather(py_v, [ic])
      dz = plsc.load_gather(pz_v, [ir]) - plsc.load_gather(pz_v, [ic])
      rowi = lane + j * L
      plsc.store_scatter(cbuf, [rowi, zero16], sv * dx)
      plsc.store_scatter(cbuf, [rowi, zero16 + 1], sv * dy)
      plsc.store_scatter(cbuf, [rowi, zero16 + 2], sv * dz)
    pltpu.sync_copy(cbuf, accc.at[idx_r2.at[i]], add=True)
    return carry

  lax.fori_loop(0, nchunk, chunk, 0)

  plsc.subcore_barrier()

  @pl.when(sid == 0)
  def _flush():
    pltpu.sync_copy(accc, aggc_hbm.at[cid])


def _sc_scatter_c(row, col, s, px, py, pz, n):
  e = row.shape[0]
  epw = e // NW
  nchunk = epw // C
  row2 = row.reshape(e // C, C)
  zc = jnp.zeros((n, PPAD), jnp.float32)
  mesh = plsc.VectorSubcoreMesh(core_axis_name="c", subcore_axis_name="s",
                                num_cores=NC, num_subcores=NS)
  f = pl.kernel(
      functools.partial(_sc_scatter_c_body, epw, nchunk),
      compiler_params=pltpu.CompilerParams(use_tc_tiling_on_sc=False,
                                           needs_layout_passes=False),
      out_type=jax.ShapeDtypeStruct((NC, n, PPAD), jnp.float32),
      mesh=mesh,
      scratch_types=(
          pltpu.VMEM((nchunk, C), jnp.int32),
          pltpu.VMEM((epw,), jnp.int32),
          pltpu.VMEM((epw,), jnp.float32),
          pltpu.VMEM((C, PPAD), jnp.float32),
          pltpu.VMEM((n,), jnp.float32),
          pltpu.VMEM((n,), jnp.float32),
          pltpu.VMEM((n,), jnp.float32),
          pltpu.VMEM_SHARED((n, PPAD), jnp.float32),
          pltpu.SemaphoreType.DMA,
      ),
  )
  return f(row2, col, s, px, py, pz, zc)


def _unpack_bf16(x32):
  lo = jax.lax.bitcast_convert_type(x32 << 16, jnp.float32)
  hi = jax.lax.bitcast_convert_type(x32 & jnp.int32(-65536), jnp.float32)
  return jnp.concatenate([lo, hi], axis=1).astype(jnp.bfloat16)


def _tc_edge_body(hr, hc, w1a, w1d, b1, w2, b2, wc1, bc1, wc2,
                  m2_out, s_out):
  hrv = hr[...]
  hcv = hc[...]
  dx = (jax.lax.bitcast_convert_type(hrv[:, 64:65], jnp.float32)
        - jax.lax.bitcast_convert_type(hcv[:, 64:65], jnp.float32))
  dy = (jax.lax.bitcast_convert_type(hrv[:, 65:66], jnp.float32)
        - jax.lax.bitcast_convert_type(hcv[:, 65:66], jnp.float32))
  dz = (jax.lax.bitcast_convert_type(hrv[:, 66:67], jnp.float32)
        - jax.lax.bitcast_convert_type(hcv[:, 66:67], jnp.float32))
  d2 = dx * dx + dy * dy + dz * dz
  dist = jnp.sqrt(d2 + 1e-8)
  hb = jnp.concatenate([_unpack_bf16(hrv[:, :64]), _unpack_bf16(hcv[:, :64])],
                       axis=1)
  x = jnp.dot(hb, w1a[...], preferred_element_type=jnp.float32)
  x = (x + dist * w1d[...] + b1[...]).astype(jnp.bfloat16)
  m1 = _silu(x)
  y = jnp.dot(m1, w2[...], preferred_element_type=jnp.float32) + b2[...]
  m2 = _silu(y)
  m2b = m2.astype(jnp.bfloat16)
  z = _silu((jnp.dot(m2b, wc1[...], preferred_element_type=jnp.float32)
             + bc1[...]).astype(jnp.bfloat16)).astype(jnp.float32)
  cw = jnp.sum(z * wc2[...], axis=1, keepdims=True)
  s = cw / (dist + 1e-8)
  m2_out[...] = m2
  s_out[...] = jnp.reshape(s, (s.shape[0],))


def _tc_edge(hr, hc, w1a, w1d, b1, w2, b2, wc1, bc1, wc2):
  e = hr.shape[0]
  be = 512
  grid = (e // be,)
  blk = lambda r, c: pl.BlockSpec((r, c), lambda i: (i, 0))
  wblk = lambda r, c: pl.BlockSpec((r, c), lambda i: (0, 0))
  return pl.pallas_call(
      _tc_edge_body,
      grid=grid,
      in_specs=[
          blk(be, HID), blk(be, HID),
          wblk(2 * HID, HID), wblk(1, HID), wblk(1, HID),
          wblk(HID, HID), wblk(1, HID),
          wblk(HID, HID), wblk(1, HID), wblk(1, HID),
      ],
      out_specs=[blk(be, HID), pl.BlockSpec((be,), lambda i: (i,))],
      out_shape=[
          jax.ShapeDtypeStruct((e, HID), jnp.float32),
          jax.ShapeDtypeStruct((e,), jnp.float32),
      ],
  )(hr, hc, w1a, w1d, b1, w2, b2, wc1, bc1, wc2)


def _tc_node_body(h, posp, aggm, aggc, wn1a, wn1b, bn1, wn2, bn2, g, b,
                  h_out, posp_out):
  agg = aggm[0]
  for p in range(1, aggm.shape[0]):
    agg = agg + aggm[p]
  x = jnp.dot(h[...], wn1a[...], preferred_element_type=jnp.float32)
  x = x + jnp.dot(agg, wn1b[...], preferred_element_type=jnp.float32)
  x = _silu(x + bn1[...])
  hupd = jnp.dot(x, wn2[...], preferred_element_type=jnp.float32) + bn2[...]
  y = h[...] + hupd
  mu = jnp.mean(y, axis=1, keepdims=True)
  var = jnp.mean((y - mu) * (y - mu), axis=1, keepdims=True)
  h_out[...] = (y - mu) * jax.lax.rsqrt(var + 1e-5) * g[...] + b[...]
  pout = posp[...]
  for p in range(aggc.shape[0]):
    pout = pout + aggc[p]
  posp_out[...] = pout


def _tc_node(h, posp, aggm, aggc, wn1a, wn1b, bn1, wn2, bn2, g, b):
  n = h.shape[0]
  npart = aggm.shape[0]
  bn = 1000
  grid = (n // bn,)
  blk = lambda r, c: pl.BlockSpec((r, c), lambda i: (i, 0))
  wblk = lambda r, c: pl.BlockSpec((r, c), lambda i: (0, 0))
  blk3 = lambda r, c: pl.BlockSpec((npart, r, c), lambda i: (0, i, 0))
  return pl.pallas_call(
      _tc_node_body,
      grid=grid,
      in_specs=[
          blk(bn, HID), blk(bn, PPAD), blk3(bn, HID), blk3(bn, PPAD),
          wblk(HID, HID), wblk(HID, HID), wblk(1, HID),
          wblk(HID, HID), wblk(1, HID), wblk(1, HID), wblk(1, HID),
      ],
      out_specs=[blk(bn, HID), blk(bn, PPAD)],
      out_shape=[
          jax.ShapeDtypeStruct((n, HID), jnp.float32),
          jax.ShapeDtypeStruct((n, PPAD), jnp.float32),
      ],
  )(h, posp, aggm, aggc, wn1a, wn1b, bn1, wn2, bn2, g, b)


@jax.jit
def kernel(h, pos, edge_index, W_e1, b_e1, W_e2, b_e2, W_n1, b_n1, W_n2,
           b_n2, W_c1, b_c1, W_c2, ln_gamma, ln_beta):
  n = h.shape[0]
  row = edge_index[0].astype(jnp.int32)
  col = edge_index[1].astype(jnp.int32)
  posp = jnp.zeros((n, PPAD), jnp.float32).at[:, :3].set(pos)

  hpacked = jax.lax.bitcast_convert_type(
      h.astype(jnp.bfloat16).reshape(n, HID // 2, 2), jnp.int32)
  posbits = jax.lax.bitcast_convert_type(pos, jnp.int32)
  tbl = jnp.concatenate(
      [hpacked, posbits, jnp.zeros((n, HID - HID // 2 - 3), jnp.int32)],
      axis=1)

  wa = W_e1[:HID]
  wb = W_e1[HID:2 * HID]
  w1a = jnp.concatenate([wa[0::2], wa[1::2], wb[0::2], wb[1::2]],
                        axis=0).astype(jnp.bfloat16)
  w1d = W_e1[2 * HID].reshape(1, HID)

  e = row.shape[0]
  quantum = NW * C
  nq = e // quantum
  splits = [q for q in (nq // 4, nq // 2, 3 * nq // 4) if 0 < q < nq]
  bounds = [0] + [q * quantum for q in splits] + [e]

  aggms = []
  aggcs = []
  for lo, hi in zip(bounds[:-1], bounds[1:]):
    rk = lax.slice_in_dim(row, lo, hi)
    ck = lax.slice_in_dim(col, lo, hi)
    hr32, hc32 = _sc_gather(tbl, rk, ck)
    m2, s = _tc_edge(hr32, hc32, w1a, w1d, b_e1.reshape(1, HID),
                     W_e2.astype(jnp.bfloat16), b_e2.reshape(1, HID),
                     W_c1.astype(jnp.bfloat16), b_c1.reshape(1, HID),
                     W_c2.reshape(1, HID))
    aggms.append(_sc_scatter_m(rk, m2, n))
    aggcs.append(_sc_scatter_c(rk, ck, s, pos[:, 0], pos[:, 1], pos[:, 2], n))

  aggm = jnp.concatenate(aggms, axis=0)
  aggc = jnp.concatenate(aggcs, axis=0)

  h_out, posp_out = _tc_node(h, posp, aggm, aggc, W_n1[:HID], W_n1[HID:],
                             b_n1.reshape(1, HID), W_n2,
                             b_n2.reshape(1, HID), ln_gamma.reshape(1, HID),
                             ln_beta.reshape(1, HID))
  return h_out, posp_out[:, :3]
